# Initial kernel scaffold; baseline (speedup 1.0000x reference)
import jax
import jax.numpy as jnp
from jax import lax
from jax.experimental import pallas as pl
from jax.experimental.pallas import tpu as pltpu
from jax.experimental.pallas import tpu_sc as plsc

N = 10000
E = 320000
D = 128
DE = 16
H = 4
O = 32
HID = H * O

NC = 2
NS = 16
NW = NC * NS
EW = E // NW
CH = 80
NCHUNK = EW // CH
RPT = N // NS
WT = 144

_mesh = plsc.VectorSubcoreMesh(core_axis_name="c", subcore_axis_name="s",
                               num_cores=NC, num_subcores=NS)


def _zero_rows(zbuf, width):
    zv = jnp.zeros((16,), jnp.float32)

    @pl.loop(0, zbuf.shape[0])
    def _(i):
        for j in range(width // 16):
            zbuf[i, pl.ds(16 * j, 16)] = zv


def _sc_degsum_body(ea_hbm, dst_hbm, easum_hbm, deg_hbm,
                    acc_ea, acc_deg, eab, onesb, idxd, zbuf):
    c = lax.axis_index("c")
    s = lax.axis_index("s")
    wid = c * NS + s

    _zero_rows(zbuf, 16)
    for k in range(RPT // zbuf.shape[0]):
        r0 = s * RPT + k * zbuf.shape[0]
        pltpu.sync_copy(zbuf, acc_ea.at[pl.ds(r0, zbuf.shape[0])])
        pltpu.sync_copy(zbuf, acc_deg.at[pl.ds(r0, zbuf.shape[0])])
    ov = jnp.ones((16,), jnp.float32)

    @pl.loop(0, CH)
    def _(i):
        onesb[i, pl.ds(0, 16)] = ov

    plsc.subcore_barrier()

    @pl.loop(0, NCHUNK)
    def _(i):
        base = wid * EW + i * CH
        pltpu.sync_copy(dst_hbm.at[pl.ds(base, CH)], idxd.at[0])
        pltpu.sync_copy(ea_hbm.at[pl.ds(base, CH)], eab)
        pltpu.sync_copy(eab, acc_ea.at[idxd.at[0]], add=True)
        pltpu.sync_copy(onesb, acc_deg.at[idxd.at[0]], add=True)

    plsc.subcore_barrier()
    r0 = s * RPT
    pltpu.sync_copy(acc_ea.at[pl.ds(r0, RPT)], easum_hbm.at[c].at[pl.ds(r0, RPT)])
    pltpu.sync_copy(acc_deg.at[pl.ds(r0, RPT)], deg_hbm.at[c].at[pl.ds(r0, RPT)])


@jax.jit
def _sc_degsum(edge_attr, dst):
    f32 = jnp.float32
    kern = pl.kernel(
        _sc_degsum_body,
        out_type=(jax.ShapeDtypeStruct((NC, N, 16), f32),
                  jax.ShapeDtypeStruct((NC, N, 16), f32)),
        mesh=_mesh,
        scratch_types=[
            pltpu.VMEM_SHARED((N, 16), f32),
            pltpu.VMEM_SHARED((N, 16), f32),
            pltpu.VMEM((CH, 16), f32),
            pltpu.VMEM((CH, 16), f32),
            pltpu.VMEM((1, CH), jnp.int32),
            pltpu.VMEM((125, 16), f32),
        ],
    )
    return kern(edge_attr, dst)


def _sc_messages_body(htab_hbm, dtab_hbm, ae_hbm, src_hbm, dst_hbm, out_hbm,
                      acc, gsrc, gdst, aeb, val, idxs, idxd, zbuf, sem1, sem2):
    c = lax.axis_index("c")
    s = lax.axis_index("s")
    wid = c * NS + s

    _zero_rows(zbuf, WT)
    for k in range(RPT // zbuf.shape[0]):
        r0 = s * RPT + k * zbuf.shape[0]
        pltpu.sync_copy(zbuf, acc.at[pl.ds(r0, zbuf.shape[0])])
    plsc.subcore_barrier()

    lane = lax.iota(jnp.int32, (16,))
    mask = jnp.where(lane < H, 1.0, 0.0).astype(jnp.float32)
    hidx = [jnp.full((16,), h, jnp.int32) for h in range(H)]

    @pl.loop(0, NCHUNK)
    def _(i):
        base = wid * EW + i * CH
        pltpu.sync_copy(src_hbm.at[pl.ds(base, CH)], idxs.at[0])
        pltpu.sync_copy(dst_hbm.at[pl.ds(base, CH)], idxd.at[0])
        pltpu.sync_copy(ae_hbm.at[pl.ds(base, CH)], aeb)
        cp1 = pltpu.async_copy(htab_hbm.at[idxs.at[0]], gsrc, sem1)
        cp2 = pltpu.async_copy(dtab_hbm.at[idxd.at[0]], gdst, sem2)
        cp1.wait()
        cp2.wait()

        @pl.loop(0, CH)
        def _(e):
            t = (gsrc[e, pl.ds(128, 16)] + gdst[e, pl.ds(0, 16)]
                 + aeb[e, pl.ds(0, 16)])
            t = jnp.maximum(t, 0.0) + 0.2 * jnp.minimum(t, 0.0)
            p = jnp.exp(t) * mask
            val[e, pl.ds(128, 16)] = p
            for h in range(H):
                pb = jnp.take(p, hidx[h], mode="promise_in_bounds")
                val[e, pl.ds(32 * h, 16)] = gsrc[e, pl.ds(32 * h, 16)] * pb
                val[e, pl.ds(32 * h + 16, 16)] = gsrc[e, pl.ds(32 * h + 16, 16)] * pb

        pltpu.sync_copy(val, acc.at[idxd.at[0]], add=True)

    plsc.subcore_barrier()
    r0 = s * RPT
    pltpu.sync_copy(acc.at[pl.ds(r0, RPT)], out_hbm.at[c].at[pl.ds(r0, RPT)])


@jax.jit
def _sc_messages(htab, dtab, ae_pad, src, dst):
    f32 = jnp.float32
    kern = pl.kernel(
        _sc_messages_body,
        out_type=jax.ShapeDtypeStruct((NC, N, WT), f32),
        mesh=_mesh,
        scratch_types=[
            pltpu.VMEM_SHARED((N, WT), f32),
            pltpu.VMEM((CH, WT), f32),
            pltpu.VMEM((CH, 16), f32),
            pltpu.VMEM((CH, 16), f32),
            pltpu.VMEM((CH, WT), f32),
            pltpu.VMEM((1, CH), jnp.int32),
            pltpu.VMEM((1, CH), jnp.int32),
            pltpu.VMEM((125, WT), f32),
            pltpu.SemaphoreType.DMA,
            pltpu.SemaphoreType.DMA,
        ],
    )
    return kern(htab, dtab, ae_pad, src, dst)


def _lrelu(x):
    return jnp.maximum(x, 0.0) + 0.2 * jnp.minimum(x, 0.0)


def _rep32(a):
    n = a.shape[0]
    return jnp.broadcast_to(a[:, :, None], (n, H, O)).reshape(n, HID)


def _node_tables(big, bs, ael):
    n = big.shape[0]
    h = big[:, :HID]
    skip = big[:, HID:2 * HID] + bs[None, :]
    a_s = big[:, 2 * HID:2 * HID + H]
    a_d = big[:, 2 * HID + H:2 * HID + 2 * H]
    p_self = jnp.exp(_lrelu(a_s + a_d + ael))
    z8 = jnp.zeros((n, 8), jnp.float32)
    htab = jnp.concatenate([h, a_s, jnp.zeros((n, 12), jnp.float32)], axis=1)
    dtab = jnp.concatenate([a_d, p_self, z8], axis=1)
    return htab, dtab, skip


def _tc_pre1_body(x_ref, es0_ref, es1_ref, dg0_ref, dg1_ref, wcat_ref,
                  weproj_ref, bs_ref, htab_ref, dtab_ref, la_ref, skip_ref):
    x = x_ref[...]
    big = jnp.dot(x, wcat_ref[...], preferred_element_type=jnp.float32)
    easum = es0_ref[...] + es1_ref[...]
    deg = dg0_ref[...] + dg1_ref[...]
    la = easum / jnp.maximum(deg, 1.0)
    ael = jnp.dot(la, weproj_ref[...], preferred_element_type=jnp.float32)
    htab, dtab, skip = _node_tables(big, bs_ref[...], ael)
    htab_ref[...] = htab
    dtab_ref[...] = dtab
    la_ref[...] = la
    skip_ref[...] = skip


@jax.jit
def _tc_pre1(x, easum, degp, wcat, weproj, bs):
    f32 = jnp.float32
    return pl.pallas_call(
        _tc_pre1_body,
        out_shape=(jax.ShapeDtypeStruct((N, WT), f32),
                   jax.ShapeDtypeStruct((N, 16), f32),
                   jax.ShapeDtypeStruct((N, 16), f32),
                   jax.ShapeDtypeStruct((N, HID), f32)),
    )(x, easum[0], easum[1], degp[0], degp[1], wcat, weproj, bs)


def _tc_ae_body(ea_ref, weproj_ref, out_ref):
    ae = jnp.dot(ea_ref[...], weproj_ref[...], preferred_element_type=jnp.float32)
    out_ref[...] = jnp.concatenate(
        [ae, jnp.zeros((ae.shape[0], 12), jnp.float32)], axis=1)


@jax.jit
def _tc_ae(edge_attr, weproj):
    be = 10000
    return pl.pallas_call(
        _tc_ae_body,
        grid=(E // be,),
        in_specs=[pl.BlockSpec((be, DE), lambda i: (i, 0)),
                  pl.BlockSpec((DE, H), lambda i: (0, 0))],
        out_specs=pl.BlockSpec((be, 16), lambda i: (i, 0)),
        out_shape=jax.ShapeDtypeStruct((E, 16), jnp.float32),
    )(edge_attr, weproj)


def _combine(p0, p1, htab, dtab, skip, b):
    P = p0 + p1
    msg = P[:, :HID]
    den = P[:, HID:HID + H]
    p_self = dtab[:, H:2 * H]
    hmat = htab[:, :HID]
    dent = den + p_self
    x1 = (msg + hmat * _rep32(p_self)) / _rep32(dent + 1e-16) + b[None, :]
    return x1 + skip


def _tc_mid_body(p0_ref, p1_ref, htab_ref, dtab_ref, skip_ref, la_ref,
                 b1_ref, g1_ref, be1_ref, wcat_ref, weproj_ref, bs2_ref,
                 htab2_ref, dtab2_ref, skip2_ref):
    t = _combine(p0_ref[...], p1_ref[...], htab_ref[...], dtab_ref[...],
                 skip_ref[...], b1_ref[...])
    mu = jnp.mean(t, axis=0, keepdims=True)
    var = jnp.mean((t - mu) ** 2, axis=0, keepdims=True)
    hn = (t - mu) / jnp.sqrt(var + 1e-5) * g1_ref[...][None, :] + be1_ref[...][None, :]
    h = jnp.where(hn > 0, hn, jnp.expm1(hn))
    big = jnp.dot(h, wcat_ref[...], preferred_element_type=jnp.float32)
    ael = jnp.dot(la_ref[...], weproj_ref[...], preferred_element_type=jnp.float32)
    htab2, dtab2, skip2 = _node_tables(big, bs2_ref[...], ael)
    htab2_ref[...] = htab2
    dtab2_ref[...] = dtab2
    skip2_ref[...] = skip2


@jax.jit
def _tc_mid(parts, htab, dtab, skip, la, b1, g1, be1, wcat2, weproj2, bs2):
    f32 = jnp.float32
    return pl.pallas_call(
        _tc_mid_body,
        out_shape=(jax.ShapeDtypeStruct((N, WT), f32),
                   jax.ShapeDtypeStruct((N, 16), f32),
                   jax.ShapeDtypeStruct((N, HID), f32)),
    )(parts[0], parts[1], htab, dtab, skip, la, b1, g1, be1, wcat2,
      weproj2, bs2)


def _tc_post2_body(p0_ref, p1_ref, htab_ref, dtab_ref, skip_ref, b2_ref,
                   wf_ref, bf_ref, out_ref):
    y = _combine(p0_ref[...], p1_ref[...], htab_ref[...], dtab_ref[...],
                 skip_ref[...], b2_ref[...])
    y = jnp.where(y > 0, y, jnp.expm1(y))
    out_ref[...] = jnp.dot(y, wf_ref[...], preferred_element_type=jnp.float32) \
        + bf_ref[...][None, :]


@jax.jit
def _tc_post2(parts, htab, dtab, skip, b2, wf, bf):
    return pl.pallas_call(
        _tc_post2_body,
        out_shape=jax.ShapeDtypeStruct((N, 1), jnp.float32),
    )(parts[0], parts[1], htab, dtab, skip, b2, wf, bf)


def kernel(x, edge_attr, edge_index, W1, att_src1, att_dst1, We1, att_e1, b1,
           Ws1, bs1, g1, be1, W2, att_src2, att_dst2, We2, att_e2, b2, Ws2,
           bs2, Wf, bf):
    f32 = jnp.float32
    src = edge_index[0].astype(jnp.int32)
    dst = edge_index[1].astype(jnp.int32)

    def proj(W, a_s, a_d, We, a_e, din):
        wsrc = (W.reshape(din, H, O) * a_s[None]).sum(-1)
        wdst = (W.reshape(din, H, O) * a_d[None]).sum(-1)
        weproj = (We.reshape(DE, H, O) * a_e[None]).sum(-1)
        return wsrc, wdst, weproj

    wsrc1, wdst1, weproj1 = proj(W1, att_src1, att_dst1, We1, att_e1, D)
    wsrc2, wdst2, weproj2 = proj(W2, att_src2, att_dst2, We2, att_e2, HID)
    wcat1 = jnp.concatenate([W1, Ws1, wsrc1, wdst1], axis=1).astype(f32)
    wcat2 = jnp.concatenate([W2, Ws2, wsrc2, wdst2], axis=1).astype(f32)

    easum, degp = _sc_degsum(edge_attr.astype(f32), dst)
    htab1, dtab1, la, skip1 = _tc_pre1(x.astype(f32), easum, degp, wcat1,
                                       weproj1, bs1)
    ae1 = _tc_ae(edge_attr.astype(f32), weproj1)
    parts1 = _sc_messages(htab1, dtab1, ae1, src, dst)
    htab2, dtab2, skip2 = _tc_mid(parts1, htab1, dtab1, skip1, la, b1, g1,
                                  be1, wcat2, weproj2, bs2)
    ae2 = _tc_ae(edge_attr.astype(f32), weproj2)
    parts2 = _sc_messages(htab2, dtab2, ae2, src, dst)
    return _tc_post2(parts2, htab2, dtab2, skip2, b2, Wf, bf)


# speedup vs baseline: 32.2372x; 32.2372x over previous
import jax
import jax.numpy as jnp
from jax import lax
from jax.experimental import pallas as pl
from jax.experimental.pallas import tpu as pltpu
from jax.experimental.pallas import tpu_sc as plsc

N = 10000
E = 320000
D = 128
DE = 16
H = 4
O = 32
HID = H * O

NC = 2
NS = 16
NW = NC * NS
EW = E // NW
CH = 80
NCHUNK = EW // CH
NP = 10240
RPT = NP // NS
WT = 144

_mesh = plsc.VectorSubcoreMesh(core_axis_name="c", subcore_axis_name="s",
                               num_cores=NC, num_subcores=NS)
_sc_params = pltpu.CompilerParams(use_tc_tiling_on_sc=False)
_tc_params = pltpu.CompilerParams(vmem_limit_bytes=100 * 1024 * 1024)


def _zero_rows(zbuf, width):
    zv = jnp.zeros((16,), jnp.float32)

    @pl.loop(0, zbuf.shape[0])
    def _(i):
        for j in range(width // 16):
            zbuf[i, pl.ds(16 * j, 16)] = zv


def _sc_degsum_body(ea_hbm, dst_hbm, easum_hbm, deg_hbm,
                    acc_ea, acc_deg, eab, onesb, idxd, zbuf):
    c = lax.axis_index("c")
    s = lax.axis_index("s")
    wid = c * NS + s

    _zero_rows(zbuf, 16)
    for k in range(RPT // zbuf.shape[0]):
        r0 = s * RPT + k * zbuf.shape[0]
        pltpu.sync_copy(zbuf, acc_ea.at[pl.ds(r0, zbuf.shape[0])])
        pltpu.sync_copy(zbuf, acc_deg.at[pl.ds(r0, zbuf.shape[0])])
    ov = jnp.ones((16,), jnp.float32)

    @pl.loop(0, CH)
    def _(i):
        onesb[i, pl.ds(0, 16)] = ov

    plsc.subcore_barrier()

    @pl.loop(0, NCHUNK)
    def _(i):
        base = wid * EW + i * CH
        pltpu.sync_copy(dst_hbm.at[pl.ds(base, CH)], idxd.at[0])
        pltpu.sync_copy(ea_hbm.at[pl.ds(base, CH)], eab)
        pltpu.sync_copy(eab, acc_ea.at[idxd.at[0]], add=True)
        pltpu.sync_copy(onesb, acc_deg.at[idxd.at[0]], add=True)

    plsc.subcore_barrier()
    r0 = s * RPT
    pltpu.sync_copy(acc_ea.at[pl.ds(r0, RPT)], easum_hbm.at[c].at[pl.ds(r0, RPT)])
    pltpu.sync_copy(acc_deg.at[pl.ds(r0, RPT)], deg_hbm.at[c].at[pl.ds(r0, RPT)])


@jax.jit
def _sc_degsum(edge_attr, dst):
    f32 = jnp.float32
    kern = pl.kernel(
        _sc_degsum_body,
        out_type=(jax.ShapeDtypeStruct((NC, NP, 16), f32),
                  jax.ShapeDtypeStruct((NC, NP, 16), f32)),
        mesh=_mesh,
        scratch_types=[
            pltpu.VMEM_SHARED((NP, 16), f32),
            pltpu.VMEM_SHARED((NP, 16), f32),
            pltpu.VMEM((CH, 16), f32),
            pltpu.VMEM((CH, 16), f32),
            pltpu.VMEM((1, CH), jnp.int32),
            pltpu.VMEM((128, 16), f32),
        ],
        compiler_params=_sc_params,
    )
    return kern(edge_attr, dst)


def _sc_messages_body(htab_hbm, dtab_hbm, ae_hbm, src_hbm, dst_hbm, out_hbm,
                      acc, gsrc, gdst, aeb, val, idxs, idxd, sem1, sem2):
    c = lax.axis_index("c")
    s = lax.axis_index("s")
    wid = c * NS + s

    _zero_rows(val, WT)
    for k in range(RPT // CH):
        r0 = s * RPT + k * CH
        pltpu.sync_copy(val, acc.at[pl.ds(r0, CH)])
    plsc.subcore_barrier()

    lane = lax.iota(jnp.int32, 16)
    mask = jnp.where(lane < H, 1.0, 0.0).astype(jnp.float32)
    hidx = [jnp.full((16, 1), h, jnp.int32) for h in range(H)]
    gdn = lax.GatherDimensionNumbers(offset_dims=(), collapsed_slice_dims=(0,),
                                     start_index_map=(0,))

    def _splat(vec, idx):
        return lax.gather(vec, idx, gdn, (1,),
                          mode=lax.GatherScatterMode.PROMISE_IN_BOUNDS)

    @pl.loop(0, NCHUNK)
    def _(i):
        base = wid * EW + i * CH
        pltpu.sync_copy(src_hbm.at[pl.ds(base, CH)], idxs.at[0])
        pltpu.sync_copy(dst_hbm.at[pl.ds(base, CH)], idxd.at[0])
        pltpu.sync_copy(ae_hbm.at[pl.ds(base, CH)], aeb)
        cp1 = pltpu.async_copy(htab_hbm.at[idxs.at[0]], gsrc, sem1)
        cp2 = pltpu.async_copy(dtab_hbm.at[idxd.at[0]], gdst, sem2)
        cp1.wait()
        cp2.wait()

        @pl.loop(0, CH)
        def _(e):
            t = (gsrc[e, pl.ds(128, 16)] + gdst[e, pl.ds(0, 16)]
                 + aeb[e, pl.ds(0, 16)])
            t = jnp.maximum(t, 0.0) + 0.2 * jnp.minimum(t, 0.0)
            p = jnp.exp(t) * mask
            val[e, pl.ds(128, 16)] = p
            for h in range(H):
                pb = _splat(p, hidx[h])
                val[e, pl.ds(32 * h, 16)] = gsrc[e, pl.ds(32 * h, 16)] * pb
                val[e, pl.ds(32 * h + 16, 16)] = gsrc[e, pl.ds(32 * h + 16, 16)] * pb

        pltpu.sync_copy(val, acc.at[idxd.at[0]], add=True)

    plsc.subcore_barrier()
    r0 = s * RPT
    pltpu.sync_copy(acc.at[pl.ds(r0, RPT)], out_hbm.at[c].at[pl.ds(r0, RPT)])


@jax.jit
def _sc_messages(htab, dtab, ae_pad, src, dst):
    f32 = jnp.float32
    kern = pl.kernel(
        _sc_messages_body,
        out_type=jax.ShapeDtypeStruct((NC, NP, WT), f32),
        mesh=_mesh,
        scratch_types=[
            pltpu.VMEM_SHARED((NP, WT), f32),
            pltpu.VMEM((CH, WT), f32),
            pltpu.VMEM((CH, 16), f32),
            pltpu.VMEM((CH, 16), f32),
            pltpu.VMEM((CH, WT), f32),
            pltpu.VMEM((1, CH), jnp.int32),
            pltpu.VMEM((1, CH), jnp.int32),
            pltpu.SemaphoreType.DMA,
            pltpu.SemaphoreType.DMA,
        ],
        compiler_params=_sc_params,
    )
    return kern(htab, dtab, ae_pad, src, dst)


BN_ = 2000
NB = N // BN_


def _lrelu(x):
    return jnp.maximum(x, 0.0) + 0.2 * jnp.minimum(x, 0.0)


def _rep32(a):
    n = a.shape[0]
    return jnp.broadcast_to(a[:, :, None], (n, H, O)).reshape(n, HID)


def _node_tables(big, bs, ael):
    n = big.shape[0]
    h = big[:, :HID]
    skip = big[:, HID:2 * HID] + bs[None, :]
    a_s = big[:, 2 * HID:2 * HID + H]
    a_d = big[:, 2 * HID + H:2 * HID + 2 * H]
    p_self = jnp.exp(_lrelu(a_s + a_d + ael))
    z8 = jnp.zeros((n, 8), jnp.float32)
    htab = jnp.concatenate([h, a_s, jnp.zeros((n, 12), jnp.float32)], axis=1)
    dtab = jnp.concatenate([a_d, p_self, z8], axis=1)
    return htab, dtab, skip


def _row_spec(w):
    return pl.BlockSpec((BN_, w), lambda i: (i, 0))


def _const_spec(shape):
    nd = len(shape)
    return pl.BlockSpec(shape, lambda i: (0,) * nd)


def _tc_pre1_body(x_ref, es0_ref, es1_ref, dg0_ref, dg1_ref, wcat_ref,
                  weproj_ref, bs_ref, htab_ref, dtab_ref, la_ref, skip_ref):
    big = jnp.dot(x_ref[...], wcat_ref[...], preferred_element_type=jnp.float32)
    easum = es0_ref[...] + es1_ref[...]
    deg = dg0_ref[...] + dg1_ref[...]
    la = easum / jnp.maximum(deg, 1.0)
    ael = jnp.dot(la, weproj_ref[...], preferred_element_type=jnp.float32)
    htab, dtab, skip = _node_tables(big, bs_ref[...], ael)
    htab_ref[...] = htab
    dtab_ref[...] = dtab
    la_ref[...] = la
    skip_ref[...] = skip


@jax.jit
def _tc_pre1(x, easum, degp, wcat, weproj, bs):
    f32 = jnp.float32
    return pl.pallas_call(
        _tc_pre1_body,
        grid=(NB,),
        in_specs=[_row_spec(D), _row_spec(16), _row_spec(16), _row_spec(16),
                  _row_spec(16), _const_spec((D, 264)), _const_spec((DE, H)),
                  _const_spec((HID,))],
        out_specs=(_row_spec(WT), _row_spec(16), _row_spec(16),
                   _row_spec(HID)),
        out_shape=(jax.ShapeDtypeStruct((N, WT), f32),
                   jax.ShapeDtypeStruct((N, 16), f32),
                   jax.ShapeDtypeStruct((N, 16), f32),
                   jax.ShapeDtypeStruct((N, HID), f32)),
    )(x, easum[0], easum[1], degp[0], degp[1], wcat, weproj, bs)


def _tc_ae_body(ea_ref, weproj_ref, out_ref):
    ae = jnp.dot(ea_ref[...], weproj_ref[...], preferred_element_type=jnp.float32)
    out_ref[...] = jnp.concatenate(
        [ae, jnp.zeros((ae.shape[0], 12), jnp.float32)], axis=1)


@jax.jit
def _tc_ae(edge_attr, weproj):
    be = 10000
    return pl.pallas_call(
        _tc_ae_body,
        grid=(E // be,),
        in_specs=[pl.BlockSpec((be, DE), lambda i: (i, 0)),
                  pl.BlockSpec((DE, H), lambda i: (0, 0))],
        out_specs=pl.BlockSpec((be, 16), lambda i: (i, 0)),
        out_shape=jax.ShapeDtypeStruct((E, 16), jnp.float32),
    )(edge_attr, weproj)


def _combine(p0, p1, htab, dtab, skip, b):
    P = p0 + p1
    msg = P[:, :HID]
    den = P[:, HID:HID + H]
    p_self = dtab[:, H:2 * H]
    hmat = htab[:, :HID]
    dent = den + p_self
    x1 = (msg + hmat * _rep32(p_self)) / _rep32(dent + 1e-16) + b[None, :]
    return x1 + skip


def _tc_combine1_body(p0_ref, p1_ref, htab_ref, dtab_ref, skip_ref, b1_ref,
                      t_ref, stats_ref):
    t = _combine(p0_ref[...], p1_ref[...], htab_ref[...], dtab_ref[...],
                 skip_ref[...], b1_ref[...])
    t_ref[...] = t
    s1 = jnp.sum(t, axis=0, keepdims=True)
    s2 = jnp.sum(t * t, axis=0, keepdims=True)
    part = jnp.concatenate([s1, s2, jnp.zeros((6, HID), jnp.float32)], axis=0)

    @pl.when(pl.program_id(0) == 0)
    def _():
        stats_ref[...] = jnp.zeros((8, HID), jnp.float32)

    stats_ref[...] += part


@jax.jit
def _tc_combine1(parts, htab, dtab, skip, b1):
    f32 = jnp.float32
    return pl.pallas_call(
        _tc_combine1_body,
        grid=(NB,),
        in_specs=[_row_spec(WT), _row_spec(WT), _row_spec(WT), _row_spec(16),
                  _row_spec(HID), _const_spec((HID,))],
        out_specs=(_row_spec(HID), _const_spec((8, HID))),
        out_shape=(jax.ShapeDtypeStruct((N, HID), f32),
                   jax.ShapeDtypeStruct((8, HID), f32)),
    )(parts[0], parts[1], htab, dtab, skip, b1)


def _tc_pre2_body(t_ref, stats_ref, g_ref, be_ref, la_ref, wcat_ref,
                  weproj_ref, bs_ref, htab_ref, dtab_ref, skip_ref):
    t = t_ref[...]
    mu = stats_ref[0:1, :] * (1.0 / N)
    var = stats_ref[1:2, :] * (1.0 / N) - mu * mu
    hn = (t - mu) / jnp.sqrt(var + 1e-5) * g_ref[...][None, :] \
        + be_ref[...][None, :]
    h = jnp.where(hn > 0, hn, jnp.exp(hn) - 1.0)
    big = jnp.dot(h, wcat_ref[...], preferred_element_type=jnp.float32)
    ael = jnp.dot(la_ref[...], weproj_ref[...], preferred_element_type=jnp.float32)
    htab, dtab, skip = _node_tables(big, bs_ref[...], ael)
    htab_ref[...] = htab
    dtab_ref[...] = dtab
    skip_ref[...] = skip


@jax.jit
def _tc_pre2(t, stats, g1, be1, la, wcat2, weproj2, bs2):
    f32 = jnp.float32
    return pl.pallas_call(
        _tc_pre2_body,
        grid=(NB,),
        in_specs=[_row_spec(HID), _const_spec((8, HID)), _const_spec((HID,)),
                  _const_spec((HID,)), _row_spec(16), _const_spec((HID, 264)),
                  _const_spec((DE, H)), _const_spec((HID,))],
        out_specs=(_row_spec(WT), _row_spec(16), _row_spec(HID)),
        out_shape=(jax.ShapeDtypeStruct((N, WT), f32),
                   jax.ShapeDtypeStruct((N, 16), f32),
                   jax.ShapeDtypeStruct((N, HID), f32)),
    )(t, stats, g1, be1, la, wcat2, weproj2, bs2)


def _tc_post2_body(p0_ref, p1_ref, htab_ref, dtab_ref, skip_ref, b2_ref,
                   wf_ref, bf_ref, out_ref):
    y = _combine(p0_ref[...], p1_ref[...], htab_ref[...], dtab_ref[...],
                 skip_ref[...], b2_ref[...])
    y = jnp.where(y > 0, y, jnp.exp(y) - 1.0)
    out_ref[...] = jnp.dot(y, wf_ref[...], preferred_element_type=jnp.float32) \
        + bf_ref[...][None, :]


@jax.jit
def _tc_post2(parts, htab, dtab, skip, b2, wf, bf):
    return pl.pallas_call(
        _tc_post2_body,
        grid=(NB,),
        in_specs=[_row_spec(WT), _row_spec(WT), _row_spec(WT), _row_spec(16),
                  _row_spec(HID), _const_spec((HID,)), _const_spec((HID, 1)),
                  _const_spec((1,))],
        out_specs=pl.BlockSpec((BN_, 1), lambda i: (i, 0)),
        out_shape=jax.ShapeDtypeStruct((N, 1), jnp.float32),
    )(parts[0], parts[1], htab, dtab, skip, b2, wf, bf)


def kernel(x, edge_attr, edge_index, W1, att_src1, att_dst1, We1, att_e1, b1,
           Ws1, bs1, g1, be1, W2, att_src2, att_dst2, We2, att_e2, b2, Ws2,
           bs2, Wf, bf):
    f32 = jnp.float32
    src = edge_index[0].astype(jnp.int32)
    dst = edge_index[1].astype(jnp.int32)

    def proj(W, a_s, a_d, We, a_e, din):
        wsrc = (W.reshape(din, H, O) * a_s[None]).sum(-1)
        wdst = (W.reshape(din, H, O) * a_d[None]).sum(-1)
        weproj = (We.reshape(DE, H, O) * a_e[None]).sum(-1)
        return wsrc, wdst, weproj

    wsrc1, wdst1, weproj1 = proj(W1, att_src1, att_dst1, We1, att_e1, D)
    wsrc2, wdst2, weproj2 = proj(W2, att_src2, att_dst2, We2, att_e2, HID)
    wcat1 = jnp.concatenate([W1, Ws1, wsrc1, wdst1], axis=1).astype(f32)
    wcat2 = jnp.concatenate([W2, Ws2, wsrc2, wdst2], axis=1).astype(f32)

    easum, degp = _sc_degsum(edge_attr.astype(f32), dst)
    htab1, dtab1, la, skip1 = _tc_pre1(x.astype(f32), easum, degp, wcat1,
                                       weproj1, bs1)
    ae1 = _tc_ae(edge_attr.astype(f32), weproj1)
    parts1 = _sc_messages(htab1, dtab1, ae1, src, dst)
    t, stats = _tc_combine1(parts1, htab1, dtab1, skip1, b1)
    htab2, dtab2, skip2 = _tc_pre2(t, stats, g1, be1, la, wcat2, weproj2, bs2)
    ae2 = _tc_ae(edge_attr.astype(f32), weproj2)
    parts2 = _sc_messages(htab2, dtab2, ae2, src, dst)
    return _tc_post2(parts2, htab2, dtab2, skip2, b2, Wf, bf)


# speedup vs baseline: 63.4123x; 1.9671x over previous
import jax
import jax.numpy as jnp
from jax import lax
from jax.experimental import pallas as pl
from jax.experimental.pallas import tpu as pltpu
from jax.experimental.pallas import tpu_sc as plsc

N = 10000
E = 320000
D = 128
DE = 16
H = 4
O = 32
HID = H * O

NC = 2
NS = 16
NW = NC * NS
EW = E // NW
CH = 80
NCHUNK = EW // CH
NP = 10240
RPT = NP // NS
WT = 144

_mesh = plsc.VectorSubcoreMesh(core_axis_name="c", subcore_axis_name="s",
                               num_cores=NC, num_subcores=NS)
_sc_params = pltpu.CompilerParams(use_tc_tiling_on_sc=False)
_tc_params = pltpu.CompilerParams(vmem_limit_bytes=100 * 1024 * 1024)


def _zero_rows(zbuf, width):
    zv = jnp.zeros((16,), jnp.float32)

    @pl.loop(0, zbuf.shape[0])
    def _(i):
        for j in range(width // 16):
            zbuf[i, pl.ds(16 * j, 16)] = zv


def _sc_degsum_body(ea_hbm, dst_hbm, easum_hbm, deg_hbm,
                    acc_ea, acc_deg, eab, onesb, idxd, zbuf):
    c = lax.axis_index("c")
    s = lax.axis_index("s")
    wid = c * NS + s

    _zero_rows(zbuf, 16)
    for k in range(RPT // zbuf.shape[0]):
        r0 = s * RPT + k * zbuf.shape[0]
        pltpu.sync_copy(zbuf, acc_ea.at[pl.ds(r0, zbuf.shape[0])])
        pltpu.sync_copy(zbuf, acc_deg.at[pl.ds(r0, zbuf.shape[0])])
    ov = jnp.ones((16,), jnp.float32)

    @pl.loop(0, CH)
    def _(i):
        onesb[i, pl.ds(0, 16)] = ov

    plsc.subcore_barrier()

    @pl.loop(0, NCHUNK)
    def _(i):
        base = wid * EW + i * CH
        pltpu.sync_copy(dst_hbm.at[pl.ds(base, CH)], idxd.at[0])
        pltpu.sync_copy(ea_hbm.at[pl.ds(base, CH)], eab)
        pltpu.sync_copy(eab, acc_ea.at[idxd.at[0]], add=True)
        pltpu.sync_copy(onesb, acc_deg.at[idxd.at[0]], add=True)

    plsc.subcore_barrier()
    r0 = s * RPT
    pltpu.sync_copy(acc_ea.at[pl.ds(r0, RPT)], easum_hbm.at[c].at[pl.ds(r0, RPT)])
    pltpu.sync_copy(acc_deg.at[pl.ds(r0, RPT)], deg_hbm.at[c].at[pl.ds(r0, RPT)])


@jax.jit
def _sc_degsum(edge_attr, dst):
    f32 = jnp.float32
    kern = pl.kernel(
        _sc_degsum_body,
        out_type=(jax.ShapeDtypeStruct((NC, NP, 16), f32),
                  jax.ShapeDtypeStruct((NC, NP, 16), f32)),
        mesh=_mesh,
        scratch_types=[
            pltpu.VMEM_SHARED((NP, 16), f32),
            pltpu.VMEM_SHARED((NP, 16), f32),
            pltpu.VMEM((CH, 16), f32),
            pltpu.VMEM((CH, 16), f32),
            pltpu.VMEM((1, CH), jnp.int32),
            pltpu.VMEM((128, 16), f32),
        ],
        compiler_params=_sc_params,
    )
    return kern(edge_attr, dst)


SB = 25
NSB = NCHUNK // SB


def _sc_messages_body(htab_hbm, dtab_hbm, ae_hbm, src_hbm, dst_hbm, out_hbm,
                      acc, gsrc0, gsrc1, gdst0, gdst1, aeb0, aeb1,
                      idxs_sb, idxd_sb, sem0, sem1):
    c = lax.axis_index("c")
    s = lax.axis_index("s")
    wid = c * NS + s

    _zero_rows(gsrc0, WT)
    for k in range(RPT // CH):
        r0 = s * RPT + k * CH
        pltpu.sync_copy(gsrc0, acc.at[pl.ds(r0, CH)])
    plsc.subcore_barrier()

    hidx = [jnp.full((16, 1), h, jnp.int32) for h in range(H)]
    gdn = lax.GatherDimensionNumbers(offset_dims=(), collapsed_slice_dims=(0,),
                                     start_index_map=(0,))

    def _splat(vec, idx):
        return lax.gather(vec, idx, gdn, (1,),
                          mode=lax.GatherScatterMode.PROMISE_IN_BOUNDS)

    bufs = ((gsrc0, gdst0, aeb0, sem0), (gsrc1, gdst1, aeb1, sem1))

    def issue(sstep, cl, b):
        gsrc, gdst, aeb, sem = bufs[b]
        base = wid * EW + sstep * (SB * CH) + cl * CH
        pltpu.async_copy(ae_hbm.at[pl.ds(base, CH)], aeb, sem)
        pltpu.async_copy(htab_hbm.at[idxs_sb.at[cl]], gsrc, sem)
        pltpu.async_copy(dtab_hbm.at[idxd_sb.at[cl]], gdst, sem)

    def consume(cl, b):
        gsrc, gdst, aeb, sem = bufs[b]
        pltpu.make_async_copy(ae_hbm.at[pl.ds(0, CH)], aeb, sem).wait()
        pltpu.make_async_copy(htab_hbm.at[idxs_sb.at[cl]], gsrc, sem).wait()
        pltpu.make_async_copy(dtab_hbm.at[idxd_sb.at[cl]], gdst, sem).wait()

        @pl.loop(0, CH, unroll=4)
        def _(e):
            t = (gsrc[e, pl.ds(128, 16)] + gdst[e, pl.ds(0, 16)]
                 + aeb[e, pl.ds(0, 16)])
            t = jnp.maximum(t, 0.0) + 0.2 * jnp.minimum(t, 0.0)
            p = jnp.exp(t)
            gsrc[e, pl.ds(128, 16)] = p
            for h in range(H):
                pb = _splat(p, hidx[h])
                gsrc[e, pl.ds(32 * h, 16)] = gsrc[e, pl.ds(32 * h, 16)] * pb
                gsrc[e, pl.ds(32 * h + 16, 16)] = \
                    gsrc[e, pl.ds(32 * h + 16, 16)] * pb

        pltpu.sync_copy(gsrc, acc.at[idxd_sb.at[cl]], add=True)

    @pl.loop(0, NSB)
    def _(sstep):
        row0 = wid * NCHUNK + sstep * SB
        pltpu.sync_copy(src_hbm.at[pl.ds(row0, SB)], idxs_sb)
        pltpu.sync_copy(dst_hbm.at[pl.ds(row0, SB)], idxd_sb)
        issue(sstep, 0, 0)

        @pl.loop(0, (SB - 1) // 2)
        def _(j):
            issue(sstep, 2 * j + 1, 1)
            consume(2 * j, 0)
            issue(sstep, 2 * j + 2, 0)
            consume(2 * j + 1, 1)

        consume(SB - 1, 0)

    plsc.subcore_barrier()
    r0 = s * RPT
    pltpu.sync_copy(acc.at[pl.ds(r0, RPT)], out_hbm.at[c].at[pl.ds(r0, RPT)])


@jax.jit
def _sc_messages(htab, dtab, ae_pad, src, dst):
    f32 = jnp.float32
    kern = pl.kernel(
        _sc_messages_body,
        out_type=jax.ShapeDtypeStruct((NC, NP, WT), f32),
        mesh=_mesh,
        scratch_types=[
            pltpu.VMEM_SHARED((NP, WT), f32),
            pltpu.VMEM((CH, WT), f32),
            pltpu.VMEM((CH, WT), f32),
            pltpu.VMEM((CH, 16), f32),
            pltpu.VMEM((CH, 16), f32),
            pltpu.VMEM((CH, 16), f32),
            pltpu.VMEM((CH, 16), f32),
            pltpu.VMEM((SB, CH), jnp.int32),
            pltpu.VMEM((SB, CH), jnp.int32),
            pltpu.SemaphoreType.DMA,
            pltpu.SemaphoreType.DMA,
        ],
        compiler_params=_sc_params,
    )
    return kern(htab, dtab, ae_pad, src.reshape(NW * NCHUNK, CH),
                dst.reshape(NW * NCHUNK, CH))


BN_ = 2000
NB = N // BN_


def _lrelu(x):
    return jnp.maximum(x, 0.0) + 0.2 * jnp.minimum(x, 0.0)


def _rep32(a):
    n = a.shape[0]
    return jnp.broadcast_to(a[:, :, None], (n, H, O)).reshape(n, HID)


def _node_tables(big, bs, ael):
    n = big.shape[0]
    h = big[:, :HID]
    skip = big[:, HID:2 * HID] + bs[None, :]
    a_s = big[:, 2 * HID:2 * HID + H]
    a_d = big[:, 2 * HID + H:2 * HID + 2 * H]
    p_self = jnp.exp(_lrelu(a_s + a_d + ael))
    z12 = jnp.zeros((n, 12), jnp.float32)
    htab = jnp.concatenate([h, a_s, z12], axis=1)
    dtab = jnp.concatenate([a_d, z12], axis=1)
    pself = jnp.concatenate([p_self, z12], axis=1)
    return htab, dtab, pself, skip


def _row_spec(w):
    return pl.BlockSpec((BN_, w), lambda i: (i, 0))


def _const_spec(shape):
    nd = len(shape)
    return pl.BlockSpec(shape, lambda i: (0,) * nd)


def _tc_pre1_body(x_ref, es0_ref, es1_ref, dg0_ref, dg1_ref, wcat_ref,
                  weproj_ref, bs_ref, htab_ref, dtab_ref, ps_ref, la_ref,
                  skip_ref):
    big = jnp.dot(x_ref[...], wcat_ref[...], preferred_element_type=jnp.float32)
    easum = es0_ref[...] + es1_ref[...]
    deg = dg0_ref[...] + dg1_ref[...]
    la = easum / jnp.maximum(deg, 1.0)
    ael = jnp.dot(la, weproj_ref[...], preferred_element_type=jnp.float32)
    htab, dtab, pself, skip = _node_tables(big, bs_ref[...], ael)
    htab_ref[...] = htab
    dtab_ref[...] = dtab
    ps_ref[...] = pself
    la_ref[...] = la
    skip_ref[...] = skip


@jax.jit
def _tc_pre1(x, easum, degp, wcat, weproj, bs):
    f32 = jnp.float32
    return pl.pallas_call(
        _tc_pre1_body,
        grid=(NB,),
        in_specs=[_row_spec(D), _row_spec(16), _row_spec(16), _row_spec(16),
                  _row_spec(16), _const_spec((D, 264)), _const_spec((DE, H)),
                  _const_spec((HID,))],
        out_specs=(_row_spec(WT), _row_spec(16), _row_spec(16), _row_spec(16),
                   _row_spec(HID)),
        out_shape=(jax.ShapeDtypeStruct((N, WT), f32),
                   jax.ShapeDtypeStruct((N, 16), f32),
                   jax.ShapeDtypeStruct((N, 16), f32),
                   jax.ShapeDtypeStruct((N, 16), f32),
                   jax.ShapeDtypeStruct((N, HID), f32)),
    )(x, easum[0], easum[1], degp[0], degp[1], wcat, weproj, bs)


def _tc_ae_body(ea_ref, weproj_ref, out_ref):
    ae = jnp.dot(ea_ref[...], weproj_ref[...], preferred_element_type=jnp.float32)
    out_ref[...] = jnp.concatenate(
        [ae, jnp.zeros((ae.shape[0], 12), jnp.float32)], axis=1)


@jax.jit
def _tc_ae(edge_attr, weproj):
    be = 10000
    return pl.pallas_call(
        _tc_ae_body,
        grid=(E // be,),
        in_specs=[pl.BlockSpec((be, DE), lambda i: (i, 0)),
                  pl.BlockSpec((DE, H), lambda i: (0, 0))],
        out_specs=pl.BlockSpec((be, 16), lambda i: (i, 0)),
        out_shape=jax.ShapeDtypeStruct((E, 16), jnp.float32),
    )(edge_attr, weproj)


def _combine(p0, p1, htab, pself, skip, b):
    P = p0 + p1
    msg = P[:, :HID]
    den = P[:, HID:HID + H]
    p_self = pself[:, :H]
    hmat = htab[:, :HID]
    dent = den + p_self
    x1 = (msg + hmat * _rep32(p_self)) / _rep32(dent + 1e-16) + b[None, :]
    return x1 + skip


def _tc_combine1_body(p0_ref, p1_ref, htab_ref, ps_ref, skip_ref, b1_ref,
                      t_ref, stats_ref):
    t = _combine(p0_ref[...], p1_ref[...], htab_ref[...], ps_ref[...],
                 skip_ref[...], b1_ref[...])
    t_ref[...] = t
    s1 = jnp.sum(t, axis=0, keepdims=True)
    s2 = jnp.sum(t * t, axis=0, keepdims=True)
    part = jnp.concatenate([s1, s2, jnp.zeros((6, HID), jnp.float32)], axis=0)

    @pl.when(pl.program_id(0) == 0)
    def _():
        stats_ref[...] = jnp.zeros((8, HID), jnp.float32)

    stats_ref[...] += part


@jax.jit
def _tc_combine1(parts, htab, pself, skip, b1):
    f32 = jnp.float32
    return pl.pallas_call(
        _tc_combine1_body,
        grid=(NB,),
        in_specs=[_row_spec(WT), _row_spec(WT), _row_spec(WT), _row_spec(16),
                  _row_spec(HID), _const_spec((HID,))],
        out_specs=(_row_spec(HID), _const_spec((8, HID))),
        out_shape=(jax.ShapeDtypeStruct((N, HID), f32),
                   jax.ShapeDtypeStruct((8, HID), f32)),
    )(parts[0], parts[1], htab, pself, skip, b1)


def _tc_pre2_body(t_ref, stats_ref, g_ref, be_ref, la_ref, wcat_ref,
                  weproj_ref, bs_ref, htab_ref, dtab_ref, ps_ref, skip_ref):
    t = t_ref[...]
    mu = stats_ref[0:1, :] * (1.0 / N)
    var = stats_ref[1:2, :] * (1.0 / N) - mu * mu
    hn = (t - mu) / jnp.sqrt(var + 1e-5) * g_ref[...][None, :] \
        + be_ref[...][None, :]
    h = jnp.where(hn > 0, hn, jnp.exp(hn) - 1.0)
    big = jnp.dot(h, wcat_ref[...], preferred_element_type=jnp.float32)
    ael = jnp.dot(la_ref[...], weproj_ref[...], preferred_element_type=jnp.float32)
    htab, dtab, pself, skip = _node_tables(big, bs_ref[...], ael)
    htab_ref[...] = htab
    dtab_ref[...] = dtab
    ps_ref[...] = pself
    skip_ref[...] = skip


@jax.jit
def _tc_pre2(t, stats, g1, be1, la, wcat2, weproj2, bs2):
    f32 = jnp.float32
    return pl.pallas_call(
        _tc_pre2_body,
        grid=(NB,),
        in_specs=[_row_spec(HID), _const_spec((8, HID)), _const_spec((HID,)),
                  _const_spec((HID,)), _row_spec(16), _const_spec((HID, 264)),
                  _const_spec((DE, H)), _const_spec((HID,))],
        out_specs=(_row_spec(WT), _row_spec(16), _row_spec(16),
                   _row_spec(HID)),
        out_shape=(jax.ShapeDtypeStruct((N, WT), f32),
                   jax.ShapeDtypeStruct((N, 16), f32),
                   jax.ShapeDtypeStruct((N, 16), f32),
                   jax.ShapeDtypeStruct((N, HID), f32)),
    )(t, stats, g1, be1, la, wcat2, weproj2, bs2)


def _tc_post2_body(p0_ref, p1_ref, htab_ref, ps_ref, skip_ref, b2_ref,
                   wf_ref, bf_ref, out_ref):
    y = _combine(p0_ref[...], p1_ref[...], htab_ref[...], ps_ref[...],
                 skip_ref[...], b2_ref[...])
    y = jnp.where(y > 0, y, jnp.exp(y) - 1.0)
    out_ref[...] = jnp.dot(y, wf_ref[...], preferred_element_type=jnp.float32) \
        + bf_ref[...][None, :]


@jax.jit
def _tc_post2(parts, htab, pself, skip, b2, wf, bf):
    return pl.pallas_call(
        _tc_post2_body,
        grid=(NB,),
        in_specs=[_row_spec(WT), _row_spec(WT), _row_spec(WT), _row_spec(16),
                  _row_spec(HID), _const_spec((HID,)), _const_spec((HID, 1)),
                  _const_spec((1,))],
        out_specs=pl.BlockSpec((BN_, 1), lambda i: (i, 0)),
        out_shape=jax.ShapeDtypeStruct((N, 1), jnp.float32),
    )(parts[0], parts[1], htab, pself, skip, b2, wf, bf)


def kernel(x, edge_attr, edge_index, W1, att_src1, att_dst1, We1, att_e1, b1,
           Ws1, bs1, g1, be1, W2, att_src2, att_dst2, We2, att_e2, b2, Ws2,
           bs2, Wf, bf):
    f32 = jnp.float32
    src = edge_index[0].astype(jnp.int32)
    dst = edge_index[1].astype(jnp.int32)

    def proj(W, a_s, a_d, We, a_e, din):
        wsrc = (W.reshape(din, H, O) * a_s[None]).sum(-1)
        wdst = (W.reshape(din, H, O) * a_d[None]).sum(-1)
        weproj = (We.reshape(DE, H, O) * a_e[None]).sum(-1)
        return wsrc, wdst, weproj

    wsrc1, wdst1, weproj1 = proj(W1, att_src1, att_dst1, We1, att_e1, D)
    wsrc2, wdst2, weproj2 = proj(W2, att_src2, att_dst2, We2, att_e2, HID)
    wcat1 = jnp.concatenate([W1, Ws1, wsrc1, wdst1], axis=1).astype(f32)
    wcat2 = jnp.concatenate([W2, Ws2, wsrc2, wdst2], axis=1).astype(f32)

    easum, degp = _sc_degsum(edge_attr.astype(f32), dst)
    htab1, dtab1, ps1, la, skip1 = _tc_pre1(x.astype(f32), easum, degp, wcat1,
                                            weproj1, bs1)
    ae1 = _tc_ae(edge_attr.astype(f32), weproj1)
    parts1 = _sc_messages(htab1, dtab1, ae1, src, dst)
    t, stats = _tc_combine1(parts1, htab1, ps1, skip1, b1)
    htab2, dtab2, ps2, skip2 = _tc_pre2(t, stats, g1, be1, la, wcat2,
                                        weproj2, bs2)
    ae2 = _tc_ae(edge_attr.astype(f32), weproj2)
    parts2 = _sc_messages(htab2, dtab2, ae2, src, dst)
    return _tc_post2(parts2, htab2, ps2, skip2, b2, Wf, bf)


# speedup vs baseline: 71.3281x; 1.1248x over previous
import jax
import jax.numpy as jnp
from jax import lax
from jax.experimental import pallas as pl
from jax.experimental.pallas import tpu as pltpu
from jax.experimental.pallas import tpu_sc as plsc

N = 10000
E = 320000
D = 128
DE = 16
H = 4
O = 32
HID = H * O

NC = 2
NS = 16
NW = NC * NS
EW = E // NW
CH = 80
NCHUNK = EW // CH
NP = 10240
RPT = NP // NS
WT = 144

_mesh = plsc.VectorSubcoreMesh(core_axis_name="c", subcore_axis_name="s",
                               num_cores=NC, num_subcores=NS)
_sc_params = pltpu.CompilerParams(use_tc_tiling_on_sc=False)
_tc_params = pltpu.CompilerParams(vmem_limit_bytes=100 * 1024 * 1024)


def _zero_rows(zbuf, width):
    zv = jnp.zeros((16,), jnp.float32)

    @pl.loop(0, zbuf.shape[0])
    def _(i):
        for j in range(width // 16):
            zbuf[i, pl.ds(16 * j, 16)] = zv


SB = 25
NSB = NCHUNK // SB


def _sc_messages_body(htab_hbm, dtab_hbm, ae_hbm, src_hbm, dst_hbm, out_hbm,
                      acc, gsrc0, gsrc1, gdst0, gdst1, aeb0, aeb1,
                      idxs_sb, idxd_sb, sem0, sem1):
    c = lax.axis_index("c")
    s = lax.axis_index("s")
    wid = c * NS + s

    _zero_rows(gsrc0, WT)
    for k in range(RPT // CH):
        r0 = s * RPT + k * CH
        pltpu.sync_copy(gsrc0, acc.at[pl.ds(r0, CH)])
    plsc.subcore_barrier()

    hidx = [jnp.full((16, 1), h, jnp.int32) for h in range(H)]
    gdn = lax.GatherDimensionNumbers(offset_dims=(), collapsed_slice_dims=(0,),
                                     start_index_map=(0,))

    def _splat(vec, idx):
        return lax.gather(vec, idx, gdn, (1,),
                          mode=lax.GatherScatterMode.PROMISE_IN_BOUNDS)

    lane = lax.iota(jnp.int32, 16)
    lt4 = jnp.where(lane < 4, 1.0, 0.0).astype(jnp.float32)
    lt8 = jnp.where(lane < 8, 1.0, 0.0).astype(jnp.float32)
    lt9 = jnp.where(lane < 9, 1.0, 0.0).astype(jnp.float32)
    lt13 = jnp.where(lane < 13, 1.0, 0.0).astype(jnp.float32)
    m1 = lt4 + (lt9 - lt8)
    m2 = (lt8 - lt4) + (lt13 - lt9)

    bufs = ((gsrc0, gdst0, aeb0, sem0), (gsrc1, gdst1, aeb1, sem1))

    def issue(sstep, cl, b):
        gsrc, gdst, aeb, sem = bufs[b]
        base = wid * EW + sstep * (SB * CH) + cl * CH
        pltpu.async_copy(ae_hbm.at[pl.ds(base, CH)], aeb, sem)
        pltpu.async_copy(htab_hbm.at[idxs_sb.at[cl]], gsrc, sem)
        pltpu.async_copy(dtab_hbm.at[idxd_sb.at[cl]], gdst, sem)

    def consume(cl, b):
        gsrc, gdst, aeb, sem = bufs[b]
        pltpu.make_async_copy(ae_hbm.at[pl.ds(0, CH)], aeb, sem).wait()
        pltpu.make_async_copy(htab_hbm.at[idxs_sb.at[cl]], gsrc, sem).wait()
        pltpu.make_async_copy(dtab_hbm.at[idxd_sb.at[cl]], gdst, sem).wait()

        @pl.loop(0, CH, unroll=4)
        def _(e):
            av = aeb[e, pl.ds(0, 16)]
            t = gsrc[e, pl.ds(128, 16)] + gdst[e, pl.ds(0, 16)] + av
            t = jnp.maximum(t, 0.0) + 0.2 * jnp.minimum(t, 0.0)
            p = jnp.exp(t)
            gsrc[e, pl.ds(128, 16)] = p * m1 + av * m2
            for h in range(H):
                pb = _splat(p, hidx[h])
                gsrc[e, pl.ds(32 * h, 16)] = gsrc[e, pl.ds(32 * h, 16)] * pb
                gsrc[e, pl.ds(32 * h + 16, 16)] = \
                    gsrc[e, pl.ds(32 * h + 16, 16)] * pb

        pltpu.sync_copy(gsrc, acc.at[idxd_sb.at[cl]], add=True)

    @pl.loop(0, NSB)
    def _(sstep):
        row0 = wid * NCHUNK + sstep * SB
        pltpu.sync_copy(src_hbm.at[pl.ds(row0, SB)], idxs_sb)
        pltpu.sync_copy(dst_hbm.at[pl.ds(row0, SB)], idxd_sb)
        issue(sstep, 0, 0)

        @pl.loop(0, (SB - 1) // 2)
        def _(j):
            issue(sstep, 2 * j + 1, 1)
            consume(2 * j, 0)
            issue(sstep, 2 * j + 2, 0)
            consume(2 * j + 1, 1)

        consume(SB - 1, 0)

    plsc.subcore_barrier()
    r0 = s * RPT
    pltpu.sync_copy(acc.at[pl.ds(r0, RPT)], out_hbm.at[c].at[pl.ds(r0, RPT)])


@jax.jit
def _sc_messages(htab, dtab, ae_pad, src, dst):
    f32 = jnp.float32
    kern = pl.kernel(
        _sc_messages_body,
        out_type=jax.ShapeDtypeStruct((NC, NP, WT), f32),
        mesh=_mesh,
        scratch_types=[
            pltpu.VMEM_SHARED((NP, WT), f32),
            pltpu.VMEM((CH, WT), f32),
            pltpu.VMEM((CH, WT), f32),
            pltpu.VMEM((CH, 16), f32),
            pltpu.VMEM((CH, 16), f32),
            pltpu.VMEM((CH, 16), f32),
            pltpu.VMEM((CH, 16), f32),
            pltpu.VMEM((SB, CH), jnp.int32),
            pltpu.VMEM((SB, CH), jnp.int32),
            pltpu.SemaphoreType.DMA,
            pltpu.SemaphoreType.DMA,
        ],
        compiler_params=_sc_params,
    )
    return kern(htab, dtab, ae_pad, src.reshape(NW * NCHUNK, CH),
                dst.reshape(NW * NCHUNK, CH))


BN_ = 2000
NB = N // BN_


def _lrelu(x):
    return jnp.maximum(x, 0.0) + 0.2 * jnp.minimum(x, 0.0)


def _rep32(a):
    n = a.shape[0]
    return jnp.broadcast_to(a[:, :, None], (n, H, O)).reshape(n, HID)


def _node_tables(big, bs, ael):
    n = big.shape[0]
    h = big[:, :HID]
    skip = big[:, HID:2 * HID] + bs[None, :]
    a_s = big[:, 2 * HID:2 * HID + H]
    a_d = big[:, 2 * HID + H:2 * HID + 2 * H]
    p_self = jnp.exp(_lrelu(a_s + a_d + ael))
    z12 = jnp.zeros((n, 12), jnp.float32)
    htab = jnp.concatenate([h, a_s, z12], axis=1)
    dtab = jnp.concatenate([a_d, z12], axis=1)
    pself = jnp.concatenate([p_self, z12], axis=1)
    return htab, dtab, pself, skip


def _row_spec(w):
    return pl.BlockSpec((BN_, w), lambda i: (i, 0))


def _const_spec(shape):
    nd = len(shape)
    return pl.BlockSpec(shape, lambda i: (0,) * nd)


def _tc_pre1_body(x_ref, wcat_ref, bs_ref, htab_ref, dtab_ref, skip_ref):
    big = jnp.dot(x_ref[...], wcat_ref[...], preferred_element_type=jnp.float32)
    n = big.shape[0]
    z12 = jnp.zeros((n, 12), jnp.float32)
    htab_ref[...] = jnp.concatenate(
        [big[:, :HID], big[:, 2 * HID:2 * HID + H], z12], axis=1)
    dtab_ref[...] = jnp.concatenate(
        [big[:, 2 * HID + H:2 * HID + 2 * H], z12], axis=1)
    skip_ref[...] = big[:, HID:2 * HID] + bs_ref[...][None, :]


@jax.jit
def _tc_pre1(x, wcat, bs):
    f32 = jnp.float32
    return pl.pallas_call(
        _tc_pre1_body,
        grid=(NB,),
        in_specs=[_row_spec(D), _const_spec((D, 264)), _const_spec((HID,))],
        out_specs=(_row_spec(WT), _row_spec(16), _row_spec(HID)),
        out_shape=(jax.ShapeDtypeStruct((N, WT), f32),
                   jax.ShapeDtypeStruct((N, 16), f32),
                   jax.ShapeDtypeStruct((N, HID), f32)),
    )(x, wcat, bs)


def _tc_ae_body(ea_ref, weproj_ref, out_ref):
    ae = jnp.dot(ea_ref[...], weproj_ref[...], preferred_element_type=jnp.float32)
    k = ae.shape[1]
    if k == 16:
        out_ref[...] = ae
    else:
        out_ref[...] = jnp.concatenate(
            [ae, jnp.zeros((ae.shape[0], 16 - k), jnp.float32)], axis=1)


@jax.jit
def _tc_ae(edge_attr, weproj):
    be = 10000
    k = weproj.shape[1]
    return pl.pallas_call(
        _tc_ae_body,
        grid=(E // be,),
        in_specs=[pl.BlockSpec((be, DE), lambda i: (i, 0)),
                  pl.BlockSpec((DE, k), lambda i: (0, 0))],
        out_specs=pl.BlockSpec((be, 16), lambda i: (i, 0)),
        out_shape=jax.ShapeDtypeStruct((E, 16), jnp.float32),
    )(edge_attr, weproj)


def _combine(p0, p1, htab, pself, skip, b):
    P = p0 + p1
    msg = P[:, :HID]
    den = P[:, HID:HID + H]
    p_self = pself[:, :H]
    hmat = htab[:, :HID]
    dent = den + p_self
    x1 = (msg + hmat * _rep32(p_self)) / _rep32(dent + 1e-16) + b[None, :]
    return x1 + skip


def _tc_combine1_body(p0_ref, p1_ref, htab_ref, dtab_ref, skip_ref, b1_ref,
                      t_ref, stats_ref, a2_ref):
    P = p0_ref[...] + p1_ref[...]
    htab = htab_ref[...]
    n = P.shape[0]
    deg = jnp.maximum(P[:, HID + 8:HID + 9], 1.0)
    ael1 = P[:, HID + 4:HID + 8] / deg
    a2_ref[...] = jnp.concatenate(
        [P[:, HID + 9:HID + 13] / deg, jnp.zeros((n, 12), jnp.float32)],
        axis=1)
    a_s = htab[:, HID:HID + H]
    a_d = dtab_ref[...][:, :H]
    p_self = jnp.exp(_lrelu(a_s + a_d + ael1))
    den = P[:, HID:HID + H] + p_self
    t = (P[:, :HID] + htab[:, :HID] * _rep32(p_self)) \
        / _rep32(den + 1e-16) + b1_ref[...][None, :] + skip_ref[...]
    t_ref[...] = t
    s1 = jnp.sum(t, axis=0, keepdims=True)
    s2 = jnp.sum(t * t, axis=0, keepdims=True)
    part = jnp.concatenate([s1, s2, jnp.zeros((6, HID), jnp.float32)], axis=0)

    @pl.when(pl.program_id(0) == 0)
    def _():
        stats_ref[...] = jnp.zeros((8, HID), jnp.float32)

    stats_ref[...] += part


@jax.jit
def _tc_combine1(parts, htab, dtab, skip, b1):
    f32 = jnp.float32
    return pl.pallas_call(
        _tc_combine1_body,
        grid=(NB,),
        in_specs=[_row_spec(WT), _row_spec(WT), _row_spec(WT), _row_spec(16),
                  _row_spec(HID), _const_spec((HID,))],
        out_specs=(_row_spec(HID), _const_spec((8, HID)), _row_spec(16)),
        out_shape=(jax.ShapeDtypeStruct((N, HID), f32),
                   jax.ShapeDtypeStruct((8, HID), f32),
                   jax.ShapeDtypeStruct((N, 16), f32)),
    )(parts[0], parts[1], htab, dtab, skip, b1)


def _tc_pre2_body(t_ref, stats_ref, g_ref, be_ref, a2_ref, wcat_ref,
                  bs_ref, htab_ref, dtab_ref, ps_ref, skip_ref):
    t = t_ref[...]
    mu = stats_ref[0:1, :] * (1.0 / N)
    var = stats_ref[1:2, :] * (1.0 / N) - mu * mu
    hn = (t - mu) / jnp.sqrt(var + 1e-5) * g_ref[...][None, :] \
        + be_ref[...][None, :]
    h = jnp.where(hn > 0, hn, jnp.exp(hn) - 1.0)
    big = jnp.dot(h, wcat_ref[...], preferred_element_type=jnp.float32)
    ael = a2_ref[...][:, :H]
    htab, dtab, pself, skip = _node_tables(big, bs_ref[...], ael)
    htab_ref[...] = htab
    dtab_ref[...] = dtab
    ps_ref[...] = pself
    skip_ref[...] = skip


@jax.jit
def _tc_pre2(t, stats, g1, be1, a2, wcat2, bs2):
    f32 = jnp.float32
    return pl.pallas_call(
        _tc_pre2_body,
        grid=(NB,),
        in_specs=[_row_spec(HID), _const_spec((8, HID)), _const_spec((HID,)),
                  _const_spec((HID,)), _row_spec(16), _const_spec((HID, 264)),
                  _const_spec((HID,))],
        out_specs=(_row_spec(WT), _row_spec(16), _row_spec(16),
                   _row_spec(HID)),
        out_shape=(jax.ShapeDtypeStruct((N, WT), f32),
                   jax.ShapeDtypeStruct((N, 16), f32),
                   jax.ShapeDtypeStruct((N, 16), f32),
                   jax.ShapeDtypeStruct((N, HID), f32)),
    )(t, stats, g1, be1, a2, wcat2, bs2)


def _tc_post2_body(p0_ref, p1_ref, htab_ref, ps_ref, skip_ref, b2_ref,
                   wf_ref, bf_ref, out_ref):
    y = _combine(p0_ref[...], p1_ref[...], htab_ref[...], ps_ref[...],
                 skip_ref[...], b2_ref[...])
    y = jnp.where(y > 0, y, jnp.exp(y) - 1.0)
    out_ref[...] = jnp.dot(y, wf_ref[...], preferred_element_type=jnp.float32) \
        + bf_ref[...][None, :]


@jax.jit
def _tc_post2(parts, htab, pself, skip, b2, wf, bf):
    return pl.pallas_call(
        _tc_post2_body,
        grid=(NB,),
        in_specs=[_row_spec(WT), _row_spec(WT), _row_spec(WT), _row_spec(16),
                  _row_spec(HID), _const_spec((HID,)), _const_spec((HID, 1)),
                  _const_spec((1,))],
        out_specs=pl.BlockSpec((BN_, 1), lambda i: (i, 0)),
        out_shape=jax.ShapeDtypeStruct((N, 1), jnp.float32),
    )(parts[0], parts[1], htab, pself, skip, b2, wf, bf)


def kernel(x, edge_attr, edge_index, W1, att_src1, att_dst1, We1, att_e1, b1,
           Ws1, bs1, g1, be1, W2, att_src2, att_dst2, We2, att_e2, b2, Ws2,
           bs2, Wf, bf):
    f32 = jnp.float32
    src = edge_index[0].astype(jnp.int32)
    dst = edge_index[1].astype(jnp.int32)

    def proj(W, a_s, a_d, We, a_e, din):
        wsrc = (W.reshape(din, H, O) * a_s[None]).sum(-1)
        wdst = (W.reshape(din, H, O) * a_d[None]).sum(-1)
        weproj = (We.reshape(DE, H, O) * a_e[None]).sum(-1)
        return wsrc, wdst, weproj

    wsrc1, wdst1, weproj1 = proj(W1, att_src1, att_dst1, We1, att_e1, D)
    wsrc2, wdst2, weproj2 = proj(W2, att_src2, att_dst2, We2, att_e2, HID)
    wcat1 = jnp.concatenate([W1, Ws1, wsrc1, wdst1], axis=1).astype(f32)
    wcat2 = jnp.concatenate([W2, Ws2, wsrc2, wdst2], axis=1).astype(f32)

    ea32 = edge_attr.astype(f32)
    htab1, dtab1, skip1 = _tc_pre1(x.astype(f32), wcat1, bs1)
    zc = jnp.zeros((DE, 1), f32)
    wp16 = jnp.concatenate([weproj1, weproj1, zc, weproj2, zc, zc, zc], axis=1)
    ae12 = _tc_ae(ea32, wp16)
    parts1 = _sc_messages(htab1, dtab1, ae12, src, dst)
    t, stats, a2 = _tc_combine1(parts1, htab1, dtab1, skip1, b1)
    htab2, dtab2, ps2, skip2 = _tc_pre2(t, stats, g1, be1, a2, wcat2, bs2)
    ae2 = _tc_ae(ea32, weproj2)
    parts2 = _sc_messages(htab2, dtab2, ae2, src, dst)
    return _tc_post2(parts2, htab2, ps2, skip2, b2, Wf, bf)


# speedup vs baseline: 75.5590x; 1.0593x over previous
import jax
import jax.numpy as jnp
from jax import lax
from jax.experimental import pallas as pl
from jax.experimental.pallas import tpu as pltpu
from jax.experimental.pallas import tpu_sc as plsc

N = 10000
E = 320000
D = 128
DE = 16
H = 4
O = 32
HID = H * O

NC = 2
NS = 16
NW = NC * NS
EW = E // NW
CH = 80
NCHUNK = EW // CH
NP = 10240
RPT = NP // NS
WT = 144

_mesh = plsc.VectorSubcoreMesh(core_axis_name="c", subcore_axis_name="s",
                               num_cores=NC, num_subcores=NS)
_sc_params = pltpu.CompilerParams(use_tc_tiling_on_sc=False)
_tc_params = pltpu.CompilerParams(vmem_limit_bytes=100 * 1024 * 1024)


def _zero_rows(zbuf, width):
    zv = jnp.zeros((16,), jnp.float32)

    @pl.loop(0, zbuf.shape[0])
    def _(i):
        for j in range(width // 16):
            zbuf[i, pl.ds(16 * j, 16)] = zv


SB = 25
NSB = NCHUNK // SB


def _sc_messages_body(htab_hbm, dtab_hbm, ae_hbm, src_hbm, dst_hbm, out_hbm,
                      acc, gsrc0, gsrc1, gdst0, gdst1, aeb0, aeb1,
                      idxs_sb, idxd_sb, sem0, sem1):
    c = lax.axis_index("c")
    s = lax.axis_index("s")
    wid = c * NS + s

    _zero_rows(gsrc0, WT)
    for k in range(RPT // CH):
        r0 = s * RPT + k * CH
        pltpu.sync_copy(gsrc0, acc.at[pl.ds(r0, CH)])
    plsc.subcore_barrier()

    hidx = [jnp.full((16, 1), h, jnp.int32) for h in range(H)]
    gdn = lax.GatherDimensionNumbers(offset_dims=(), collapsed_slice_dims=(0,),
                                     start_index_map=(0,))

    def _splat(vec, idx):
        return lax.gather(vec, idx, gdn, (1,),
                          mode=lax.GatherScatterMode.PROMISE_IN_BOUNDS)

    lane = lax.iota(jnp.int32, 16)
    lt4 = jnp.where(lane < 4, 1.0, 0.0).astype(jnp.float32)
    lt8 = jnp.where(lane < 8, 1.0, 0.0).astype(jnp.float32)
    lt9 = jnp.where(lane < 9, 1.0, 0.0).astype(jnp.float32)
    lt13 = jnp.where(lane < 13, 1.0, 0.0).astype(jnp.float32)
    m1 = lt4 + (lt9 - lt8)
    m2 = (lt8 - lt4) + (lt13 - lt9)

    bufs = ((gsrc0, gdst0, aeb0, sem0), (gsrc1, gdst1, aeb1, sem1))
    CR = CH // 8

    def issue(sstep, cl, b):
        gsrc, gdst, aeb, sem = bufs[b]
        base = wid * EW + sstep * (SB * CH) + cl * CH
        pltpu.async_copy(ae_hbm.at[pl.ds(base // 8, CR)], aeb, sem)
        pltpu.async_copy(htab_hbm.at[idxs_sb.at[cl]], gsrc, sem)
        pltpu.async_copy(dtab_hbm.at[idxd_sb.at[cl]], gdst, sem)

    def consume(cl, b):
        gsrc, gdst, aeb, sem = bufs[b]
        pltpu.make_async_copy(ae_hbm.at[pl.ds(0, CR)], aeb, sem).wait()
        pltpu.make_async_copy(htab_hbm.at[idxs_sb.at[cl]], gsrc, sem).wait()
        pltpu.make_async_copy(dtab_hbm.at[idxd_sb.at[cl]], gdst, sem).wait()

        @pl.loop(0, CR, unroll=2)
        def _(r):
            for j in range(8):
                e = r * 8 + j
                av = aeb[r, pl.ds(16 * j, 16)]
                t = gsrc[e, pl.ds(128, 16)] + gdst[e, pl.ds(0, 16)] + av
                t = jnp.maximum(t, 0.0) + 0.2 * jnp.minimum(t, 0.0)
                p = jnp.exp(t)
                gsrc[e, pl.ds(128, 16)] = p * m1 + av * m2
                for h in range(H):
                    pb = _splat(p, hidx[h])
                    gsrc[e, pl.ds(32 * h, 16)] = gsrc[e, pl.ds(32 * h, 16)] * pb
                    gsrc[e, pl.ds(32 * h + 16, 16)] = \
                        gsrc[e, pl.ds(32 * h + 16, 16)] * pb

        pltpu.sync_copy(gsrc, acc.at[idxd_sb.at[cl]], add=True)

    @pl.loop(0, NSB)
    def _(sstep):
        row0 = wid * NCHUNK + sstep * SB
        pltpu.sync_copy(src_hbm.at[pl.ds(row0, SB)], idxs_sb)
        pltpu.sync_copy(dst_hbm.at[pl.ds(row0, SB)], idxd_sb)
        issue(sstep, 0, 0)

        @pl.loop(0, (SB - 1) // 2)
        def _(j):
            issue(sstep, 2 * j + 1, 1)
            consume(2 * j, 0)
            issue(sstep, 2 * j + 2, 0)
            consume(2 * j + 1, 1)

        consume(SB - 1, 0)

    plsc.subcore_barrier()
    r0 = s * RPT
    pltpu.sync_copy(acc.at[pl.ds(r0, RPT)], out_hbm.at[c].at[pl.ds(r0, RPT)])


@jax.jit
def _sc_messages(htab, dtab, ae_pad, src, dst):
    f32 = jnp.float32
    kern = pl.kernel(
        _sc_messages_body,
        out_type=jax.ShapeDtypeStruct((NC, NP, WT), f32),
        mesh=_mesh,
        scratch_types=[
            pltpu.VMEM_SHARED((NP, WT), f32),
            pltpu.VMEM((CH, WT), f32),
            pltpu.VMEM((CH, WT), f32),
            pltpu.VMEM((CH, 16), f32),
            pltpu.VMEM((CH, 16), f32),
            pltpu.VMEM((CH // 8, 128), f32),
            pltpu.VMEM((CH // 8, 128), f32),
            pltpu.VMEM((SB, CH), jnp.int32),
            pltpu.VMEM((SB, CH), jnp.int32),
            pltpu.SemaphoreType.DMA,
            pltpu.SemaphoreType.DMA,
        ],
        compiler_params=_sc_params,
    )
    return kern(htab, dtab, ae_pad, src.reshape(NW * NCHUNK, CH),
                dst.reshape(NW * NCHUNK, CH))


BN_ = 2000
NB = N // BN_


def _lrelu(x):
    return jnp.maximum(x, 0.0) + 0.2 * jnp.minimum(x, 0.0)


def _rep32(a):
    n = a.shape[0]
    return jnp.broadcast_to(a[:, :, None], (n, H, O)).reshape(n, HID)


def _node_tables(big, bs, ael):
    n = big.shape[0]
    h = big[:, :HID]
    skip = big[:, HID:2 * HID] + bs[None, :]
    a_s = big[:, 2 * HID:2 * HID + H]
    a_d = big[:, 2 * HID + H:2 * HID + 2 * H]
    p_self = jnp.exp(_lrelu(a_s + a_d + ael))
    z12 = jnp.zeros((n, 12), jnp.float32)
    htab = jnp.concatenate([h, a_s, z12], axis=1)
    dtab = jnp.concatenate([a_d, z12], axis=1)
    pself = jnp.concatenate([p_self, z12], axis=1)
    return htab, dtab, pself, skip


def _row_spec(w):
    return pl.BlockSpec((BN_, w), lambda i: (i, 0))


def _const_spec(shape):
    nd = len(shape)
    return pl.BlockSpec(shape, lambda i: (0,) * nd)


def _tc_pre1_body(x_ref, wcat_ref, bs_ref, htab_ref, dtab_ref, skip_ref):
    big = jnp.dot(x_ref[...], wcat_ref[...], preferred_element_type=jnp.float32)
    n = big.shape[0]
    z12 = jnp.zeros((n, 12), jnp.float32)
    htab_ref[...] = jnp.concatenate(
        [big[:, :HID], big[:, 2 * HID:2 * HID + H], z12], axis=1)
    dtab_ref[...] = jnp.concatenate(
        [big[:, 2 * HID + H:2 * HID + 2 * H], z12], axis=1)
    skip_ref[...] = big[:, HID:2 * HID] + bs_ref[...][None, :]


@jax.jit
def _tc_pre1(x, wcat, bs):
    f32 = jnp.float32
    return pl.pallas_call(
        _tc_pre1_body,
        grid=(NB,),
        in_specs=[_row_spec(D), _const_spec((D, 264)), _const_spec((HID,))],
        out_specs=(_row_spec(WT), _row_spec(16), _row_spec(HID)),
        out_shape=(jax.ShapeDtypeStruct((N, WT), f32),
                   jax.ShapeDtypeStruct((N, 16), f32),
                   jax.ShapeDtypeStruct((N, HID), f32)),
    )(x, wcat, bs)


def _tc_ae_body(ea_ref, wpa_ref, wpb_ref, outa_ref, outb_ref):
    ea = ea_ref[...]
    q = ea.shape[0] // 8
    aea = jnp.dot(ea, wpa_ref[...], preferred_element_type=jnp.float32)
    aeb = jnp.dot(ea, wpb_ref[...], preferred_element_type=jnp.float32)
    outa_ref[...] = jnp.concatenate(
        [aea[j * q:(j + 1) * q, :] for j in range(8)], axis=1)
    outb_ref[...] = jnp.concatenate(
        [aeb[j * q:(j + 1) * q, :] for j in range(8)], axis=1)


@jax.jit
def _tc_ae(edge_attr, wpa, wpb):
    be = 8000
    spec = pl.BlockSpec((be // 8, 128), lambda i: (i, 0))
    shp = jax.ShapeDtypeStruct((E // 8, 128), jnp.float32)
    return pl.pallas_call(
        _tc_ae_body,
        grid=(E // be,),
        in_specs=[pl.BlockSpec((be, DE), lambda i: (i, 0)),
                  _const_spec((DE, 16)), _const_spec((DE, 16))],
        out_specs=(spec, spec),
        out_shape=(shp, shp),
    )(edge_attr, wpa, wpb)


def _combine(p0, p1, htab, pself, skip, b):
    P = p0 + p1
    msg = P[:, :HID]
    den = P[:, HID:HID + H]
    p_self = pself[:, :H]
    hmat = htab[:, :HID]
    dent = den + p_self
    x1 = (msg + hmat * _rep32(p_self)) / _rep32(dent + 1e-16) + b[None, :]
    return x1 + skip


def _tc_combine1_body(p0_ref, p1_ref, htab_ref, dtab_ref, skip_ref, b1_ref,
                      t_ref, stats_ref, a2_ref):
    P = p0_ref[...] + p1_ref[...]
    htab = htab_ref[...]
    n = P.shape[0]
    deg = jnp.maximum(P[:, HID + 8:HID + 9], 1.0)
    ael1 = P[:, HID + 4:HID + 8] / deg
    a2_ref[...] = jnp.concatenate(
        [P[:, HID + 9:HID + 13] / deg, jnp.zeros((n, 12), jnp.float32)],
        axis=1)
    a_s = htab[:, HID:HID + H]
    a_d = dtab_ref[...][:, :H]
    p_self = jnp.exp(_lrelu(a_s + a_d + ael1))
    den = P[:, HID:HID + H] + p_self
    t = (P[:, :HID] + htab[:, :HID] * _rep32(p_self)) \
        / _rep32(den + 1e-16) + b1_ref[...][None, :] + skip_ref[...]
    t_ref[...] = t
    s1 = jnp.sum(t, axis=0, keepdims=True)
    s2 = jnp.sum(t * t, axis=0, keepdims=True)
    part = jnp.concatenate([s1, s2, jnp.zeros((6, HID), jnp.float32)], axis=0)

    @pl.when(pl.program_id(0) == 0)
    def _():
        stats_ref[...] = jnp.zeros((8, HID), jnp.float32)

    stats_ref[...] += part


@jax.jit
def _tc_combine1(parts, htab, dtab, skip, b1):
    f32 = jnp.float32
    return pl.pallas_call(
        _tc_combine1_body,
        grid=(NB,),
        in_specs=[_row_spec(WT), _row_spec(WT), _row_spec(WT), _row_spec(16),
                  _row_spec(HID), _const_spec((HID,))],
        out_specs=(_row_spec(HID), _const_spec((8, HID)), _row_spec(16)),
        out_shape=(jax.ShapeDtypeStruct((N, HID), f32),
                   jax.ShapeDtypeStruct((8, HID), f32),
                   jax.ShapeDtypeStruct((N, 16), f32)),
    )(parts[0], parts[1], htab, dtab, skip, b1)


def _tc_pre2_body(t_ref, stats_ref, g_ref, be_ref, a2_ref, wcat_ref,
                  bs_ref, htab_ref, dtab_ref, ps_ref, skip_ref):
    t = t_ref[...]
    mu = stats_ref[0:1, :] * (1.0 / N)
    var = stats_ref[1:2, :] * (1.0 / N) - mu * mu
    hn = (t - mu) / jnp.sqrt(var + 1e-5) * g_ref[...][None, :] \
        + be_ref[...][None, :]
    h = jnp.where(hn > 0, hn, jnp.exp(hn) - 1.0)
    big = jnp.dot(h, wcat_ref[...], preferred_element_type=jnp.float32)
    ael = a2_ref[...][:, :H]
    htab, dtab, pself, skip = _node_tables(big, bs_ref[...], ael)
    htab_ref[...] = htab
    dtab_ref[...] = dtab
    ps_ref[...] = pself
    skip_ref[...] = skip


@jax.jit
def _tc_pre2(t, stats, g1, be1, a2, wcat2, bs2):
    f32 = jnp.float32
    return pl.pallas_call(
        _tc_pre2_body,
        grid=(NB,),
        in_specs=[_row_spec(HID), _const_spec((8, HID)), _const_spec((HID,)),
                  _const_spec((HID,)), _row_spec(16), _const_spec((HID, 264)),
                  _const_spec((HID,))],
        out_specs=(_row_spec(WT), _row_spec(16), _row_spec(16),
                   _row_spec(HID)),
        out_shape=(jax.ShapeDtypeStruct((N, WT), f32),
                   jax.ShapeDtypeStruct((N, 16), f32),
                   jax.ShapeDtypeStruct((N, 16), f32),
                   jax.ShapeDtypeStruct((N, HID), f32)),
    )(t, stats, g1, be1, a2, wcat2, bs2)


def _tc_post2_body(p0_ref, p1_ref, htab_ref, ps_ref, skip_ref, b2_ref,
                   wf_ref, bf_ref, out_ref):
    y = _combine(p0_ref[...], p1_ref[...], htab_ref[...], ps_ref[...],
                 skip_ref[...], b2_ref[...])
    y = jnp.where(y > 0, y, jnp.exp(y) - 1.0)
    out_ref[...] = jnp.dot(y, wf_ref[...], preferred_element_type=jnp.float32) \
        + bf_ref[...][None, :]


@jax.jit
def _tc_post2(parts, htab, pself, skip, b2, wf, bf):
    return pl.pallas_call(
        _tc_post2_body,
        grid=(NB,),
        in_specs=[_row_spec(WT), _row_spec(WT), _row_spec(WT), _row_spec(16),
                  _row_spec(HID), _const_spec((HID,)), _const_spec((HID, 1)),
                  _const_spec((1,))],
        out_specs=pl.BlockSpec((BN_, 1), lambda i: (i, 0)),
        out_shape=jax.ShapeDtypeStruct((N, 1), jnp.float32),
    )(parts[0], parts[1], htab, pself, skip, b2, wf, bf)


def kernel(x, edge_attr, edge_index, W1, att_src1, att_dst1, We1, att_e1, b1,
           Ws1, bs1, g1, be1, W2, att_src2, att_dst2, We2, att_e2, b2, Ws2,
           bs2, Wf, bf):
    f32 = jnp.float32
    src = edge_index[0].astype(jnp.int32)
    dst = edge_index[1].astype(jnp.int32)

    def proj(W, a_s, a_d, We, a_e, din):
        wsrc = (W.reshape(din, H, O) * a_s[None]).sum(-1)
        wdst = (W.reshape(din, H, O) * a_d[None]).sum(-1)
        weproj = (We.reshape(DE, H, O) * a_e[None]).sum(-1)
        return wsrc, wdst, weproj

    wsrc1, wdst1, weproj1 = proj(W1, att_src1, att_dst1, We1, att_e1, D)
    wsrc2, wdst2, weproj2 = proj(W2, att_src2, att_dst2, We2, att_e2, HID)
    wcat1 = jnp.concatenate([W1, Ws1, wsrc1, wdst1], axis=1).astype(f32)
    wcat2 = jnp.concatenate([W2, Ws2, wsrc2, wdst2], axis=1).astype(f32)

    ea32 = edge_attr.astype(f32)
    htab1, dtab1, skip1 = _tc_pre1(x.astype(f32), wcat1, bs1)
    zc = jnp.zeros((DE, 1), f32)
    wp16 = jnp.concatenate([weproj1, weproj1, zc, weproj2, zc, zc, zc], axis=1)
    wp2p = jnp.concatenate([weproj2] + [zc] * 12, axis=1)
    ae12, ae2 = _tc_ae(ea32, wp16, wp2p)
    src = src.reshape(E // 8000, 8, 1000).transpose(0, 2, 1).reshape(-1)
    dst = dst.reshape(E // 8000, 8, 1000).transpose(0, 2, 1).reshape(-1)
    parts1 = _sc_messages(htab1, dtab1, ae12, src, dst)
    t, stats, a2 = _tc_combine1(parts1, htab1, dtab1, skip1, b1)
    htab2, dtab2, ps2, skip2 = _tc_pre2(t, stats, g1, be1, a2, wcat2, bs2)
    parts2 = _sc_messages(htab2, dtab2, ae2, src, dst)
    return _tc_post2(parts2, htab2, ps2, skip2, b2, Wf, bf)


# speedup vs baseline: 82.8642x; 1.0967x over previous
import jax
import jax.numpy as jnp
from jax import lax
from jax.experimental import pallas as pl
from jax.experimental.pallas import tpu as pltpu
from jax.experimental.pallas import tpu_sc as plsc

N = 10000
E = 320000
D = 128
DE = 16
H = 4
O = 32
HID = H * O

NC = 2
NS = 16
NW = NC * NS
EW = E // NW
CH = 80
NCHUNK = EW // CH
NP = 10240
RPT = NP // NS
WT = 144

_mesh = plsc.VectorSubcoreMesh(core_axis_name="c", subcore_axis_name="s",
                               num_cores=NC, num_subcores=NS)
_sc_params = pltpu.CompilerParams(use_tc_tiling_on_sc=False)
_tc_params = pltpu.CompilerParams(vmem_limit_bytes=100 * 1024 * 1024)


def _zero_rows(zbuf, width):
    zv = jnp.zeros((16,), jnp.float32)

    @pl.loop(0, zbuf.shape[0])
    def _(i):
        for j in range(width // 16):
            zbuf[i, pl.ds(16 * j, 16)] = zv


SB = 25
NSB = NCHUNK // SB


def _sc_messages_body(htab_hbm, dtab_hbm, ae_hbm, src_hbm, dst_hbm, out_hbm,
                      acc, gsrc0, gsrc1, gdst0, gdst1, aeb0, aeb1,
                      idxs_sb, idxd_sb, sem0, sem1):
    c = lax.axis_index("c")
    s = lax.axis_index("s")
    wid = c * NS + s

    _zero_rows(gsrc0, WT)
    for k in range(RPT // CH):
        r0 = s * RPT + k * CH
        pltpu.sync_copy(gsrc0, acc.at[pl.ds(r0, CH)])
    plsc.subcore_barrier()

    hidx = [jnp.full((16, 1), h, jnp.int32) for h in range(H)]
    gdn = lax.GatherDimensionNumbers(offset_dims=(), collapsed_slice_dims=(0,),
                                     start_index_map=(0,))

    def _splat(vec, idx):
        return lax.gather(vec, idx, gdn, (1,),
                          mode=lax.GatherScatterMode.PROMISE_IN_BOUNDS)

    lane = lax.iota(jnp.int32, 16)
    lt4 = jnp.where(lane < 4, 1.0, 0.0).astype(jnp.float32)
    lt8 = jnp.where(lane < 8, 1.0, 0.0).astype(jnp.float32)
    lt9 = jnp.where(lane < 9, 1.0, 0.0).astype(jnp.float32)
    lt13 = jnp.where(lane < 13, 1.0, 0.0).astype(jnp.float32)
    m1 = lt4 + (lt9 - lt8)
    m2 = (lt8 - lt4) + (lt13 - lt9)

    bufs = ((gsrc0, gdst0, aeb0, sem0), (gsrc1, gdst1, aeb1, sem1))
    CR = CH // 8

    def issue(sstep, cl, b):
        gsrc, gdst, aeb, sem = bufs[b]
        base = wid * EW + sstep * (SB * CH) + cl * CH
        pltpu.async_copy(ae_hbm.at[pl.ds(base // 8, CR)], aeb, sem)
        pltpu.async_copy(htab_hbm.at[idxs_sb.at[cl]], gsrc, sem)
        pltpu.async_copy(dtab_hbm.at[idxd_sb.at[cl]], gdst, sem)

    def consume(cl, b):
        gsrc, gdst, aeb, sem = bufs[b]
        pltpu.make_async_copy(ae_hbm.at[pl.ds(0, CR)], aeb, sem).wait()
        pltpu.make_async_copy(htab_hbm.at[idxs_sb.at[cl]], gsrc, sem).wait()
        pltpu.make_async_copy(dtab_hbm.at[idxd_sb.at[cl]], gdst, sem).wait()

        @pl.loop(0, CR, unroll=2)
        def _(r):
            for j in range(8):
                e = r * 8 + j
                av = aeb[r, pl.ds(16 * j, 16)]
                t = gsrc[e, pl.ds(128, 16)] + gdst[e, pl.ds(0, 16)] + av
                t = jnp.maximum(t, 0.0) + 0.2 * jnp.minimum(t, 0.0)
                p = jnp.exp(t)
                gsrc[e, pl.ds(128, 16)] = p * m1 + av * m2
                for h in range(H):
                    pb = _splat(p, hidx[h])
                    gsrc[e, pl.ds(32 * h, 16)] = gsrc[e, pl.ds(32 * h, 16)] * pb
                    gsrc[e, pl.ds(32 * h + 16, 16)] = \
                        gsrc[e, pl.ds(32 * h + 16, 16)] * pb

        pltpu.sync_copy(gsrc, acc.at[idxd_sb.at[cl]], add=True)

    @pl.loop(0, NSB)
    def _(sstep):
        row0 = wid * NCHUNK + sstep * SB
        pltpu.sync_copy(src_hbm.at[pl.ds(row0, SB)], idxs_sb)
        pltpu.sync_copy(dst_hbm.at[pl.ds(row0, SB)], idxd_sb)
        issue(sstep, 0, 0)

        @pl.loop(0, (SB - 1) // 2)
        def _(j):
            issue(sstep, 2 * j + 1, 1)
            consume(2 * j, 0)
            issue(sstep, 2 * j + 2, 0)
            consume(2 * j + 1, 1)

        consume(SB - 1, 0)

    plsc.subcore_barrier()
    r0 = s * RPT
    pltpu.sync_copy(acc.at[pl.ds(r0, RPT)], out_hbm.at[c].at[pl.ds(r0, RPT)])


@jax.jit
def _sc_messages(htab, dtab, ae_pad, src, dst):
    f32 = jnp.float32
    kern = pl.kernel(
        _sc_messages_body,
        out_type=jax.ShapeDtypeStruct((NC, NP, WT), f32),
        mesh=_mesh,
        scratch_types=[
            pltpu.VMEM_SHARED((NP, WT), f32),
            pltpu.VMEM((CH, WT), f32),
            pltpu.VMEM((CH, WT), f32),
            pltpu.VMEM((CH, 16), f32),
            pltpu.VMEM((CH, 16), f32),
            pltpu.VMEM((CH // 8, 128), f32),
            pltpu.VMEM((CH // 8, 128), f32),
            pltpu.VMEM((SB, CH), jnp.int32),
            pltpu.VMEM((SB, CH), jnp.int32),
            pltpu.SemaphoreType.DMA,
            pltpu.SemaphoreType.DMA,
        ],
        compiler_params=_sc_params,
    )
    return kern(htab, dtab, ae_pad, src.reshape(NW * NCHUNK, CH),
                dst.reshape(NW * NCHUNK, CH))


BN_ = 2000
NB = N // BN_


def _lrelu(x):
    return jnp.maximum(x, 0.0) + 0.2 * jnp.minimum(x, 0.0)


def _rep32(a):
    n = a.shape[0]
    return jnp.broadcast_to(a[:, :, None], (n, H, O)).reshape(n, HID)


def _node_tables(big, bs, ael):
    n = big.shape[0]
    h = big[:, :HID]
    skip = big[:, HID:2 * HID] + bs[None, :]
    a_s = big[:, 2 * HID:2 * HID + H]
    a_d = big[:, 2 * HID + H:2 * HID + 2 * H]
    p_self = jnp.exp(_lrelu(a_s + a_d + ael))
    z12 = jnp.zeros((n, 12), jnp.float32)
    htab = jnp.concatenate([h, a_s, z12], axis=1)
    dtab = jnp.concatenate([a_d, z12], axis=1)
    pself = jnp.concatenate([p_self, z12], axis=1)
    return htab, dtab, pself, skip


def _row_spec(w):
    return pl.BlockSpec((BN_, w), lambda i: (i, 0))


def _const_spec(shape):
    nd = len(shape)
    return pl.BlockSpec(shape, lambda i: (0,) * nd)


def _tc_pre1_body(x_ref, wcat_ref, bs_ref, htab_ref, dtab_ref, skip_ref):
    big = jnp.dot(x_ref[...], wcat_ref[...], preferred_element_type=jnp.float32)
    n = big.shape[0]
    z12 = jnp.zeros((n, 12), jnp.float32)
    htab_ref[...] = jnp.concatenate(
        [big[:, :HID], big[:, 2 * HID:2 * HID + H], z12], axis=1)
    dtab_ref[...] = jnp.concatenate(
        [big[:, 2 * HID + H:2 * HID + 2 * H], z12], axis=1)
    skip_ref[...] = big[:, HID:2 * HID] + bs_ref[...][None, :]


@jax.jit
def _tc_pre1(x, wcat, bs):
    f32 = jnp.float32
    return pl.pallas_call(
        _tc_pre1_body,
        grid=(NB,),
        in_specs=[_row_spec(D), _const_spec((D, 264)), _const_spec((HID,))],
        out_specs=(_row_spec(WT), _row_spec(16), _row_spec(HID)),
        out_shape=(jax.ShapeDtypeStruct((N, WT), f32),
                   jax.ShapeDtypeStruct((N, 16), f32),
                   jax.ShapeDtypeStruct((N, HID), f32)),
    )(x, wcat, bs)


def _tc_ae_body(ea_ref, wpa_ref, wpb_ref, outa_ref, outb_ref):
    ea = ea_ref[...]
    outa_ref[...] = jnp.dot(ea, wpa_ref[...],
                            preferred_element_type=jnp.float32)
    outb_ref[...] = jnp.dot(ea, wpb_ref[...],
                            preferred_element_type=jnp.float32)


@jax.jit
def _tc_ae(ea128, wpa, wpb):
    br = 4000
    spec = pl.BlockSpec((br, 128), lambda i: (i, 0))
    shp = jax.ShapeDtypeStruct((E // 8, 128), jnp.float32)
    return pl.pallas_call(
        _tc_ae_body,
        grid=(E // 8 // br,),
        in_specs=[spec, _const_spec((128, 128)), _const_spec((128, 128))],
        out_specs=(spec, spec),
        out_shape=(shp, shp),
    )(ea128, wpa, wpb)


def _combine(p0, p1, htab, pself, skip, b):
    P = p0 + p1
    msg = P[:, :HID]
    den = P[:, HID:HID + H]
    p_self = pself[:, :H]
    hmat = htab[:, :HID]
    dent = den + p_self
    x1 = (msg + hmat * _rep32(p_self)) / _rep32(dent + 1e-16) + b[None, :]
    return x1 + skip


def _tc_combine1_body(p0_ref, p1_ref, htab_ref, dtab_ref, skip_ref, b1_ref,
                      t_ref, stats_ref, a2_ref):
    P = p0_ref[...] + p1_ref[...]
    htab = htab_ref[...]
    n = P.shape[0]
    deg = jnp.maximum(P[:, HID + 8:HID + 9], 1.0)
    ael1 = P[:, HID + 4:HID + 8] / deg
    a2_ref[...] = jnp.concatenate(
        [P[:, HID + 9:HID + 13] / deg, jnp.zeros((n, 12), jnp.float32)],
        axis=1)
    a_s = htab[:, HID:HID + H]
    a_d = dtab_ref[...][:, :H]
    p_self = jnp.exp(_lrelu(a_s + a_d + ael1))
    den = P[:, HID:HID + H] + p_self
    t = (P[:, :HID] + htab[:, :HID] * _rep32(p_self)) \
        / _rep32(den + 1e-16) + b1_ref[...][None, :] + skip_ref[...]
    t_ref[...] = t
    s1 = jnp.sum(t, axis=0, keepdims=True)
    s2 = jnp.sum(t * t, axis=0, keepdims=True)
    part = jnp.concatenate([s1, s2, jnp.zeros((6, HID), jnp.float32)], axis=0)

    @pl.when(pl.program_id(0) == 0)
    def _():
        stats_ref[...] = jnp.zeros((8, HID), jnp.float32)

    stats_ref[...] += part


@jax.jit
def _tc_combine1(parts, htab, dtab, skip, b1):
    f32 = jnp.float32
    return pl.pallas_call(
        _tc_combine1_body,
        grid=(NB,),
        in_specs=[_row_spec(WT), _row_spec(WT), _row_spec(WT), _row_spec(16),
                  _row_spec(HID), _const_spec((HID,))],
        out_specs=(_row_spec(HID), _const_spec((8, HID)), _row_spec(16)),
        out_shape=(jax.ShapeDtypeStruct((N, HID), f32),
                   jax.ShapeDtypeStruct((8, HID), f32),
                   jax.ShapeDtypeStruct((N, 16), f32)),
    )(parts[0], parts[1], htab, dtab, skip, b1)


def _tc_pre2_body(t_ref, stats_ref, g_ref, be_ref, a2_ref, wcat_ref,
                  bs_ref, htab_ref, dtab_ref, ps_ref, skip_ref):
    t = t_ref[...]
    mu = stats_ref[0:1, :] * (1.0 / N)
    var = stats_ref[1:2, :] * (1.0 / N) - mu * mu
    hn = (t - mu) / jnp.sqrt(var + 1e-5) * g_ref[...][None, :] \
        + be_ref[...][None, :]
    h = jnp.where(hn > 0, hn, jnp.exp(hn) - 1.0)
    big = jnp.dot(h, wcat_ref[...], preferred_element_type=jnp.float32)
    ael = a2_ref[...][:, :H]
    htab, dtab, pself, skip = _node_tables(big, bs_ref[...], ael)
    htab_ref[...] = htab
    dtab_ref[...] = dtab
    ps_ref[...] = pself
    skip_ref[...] = skip


@jax.jit
def _tc_pre2(t, stats, g1, be1, a2, wcat2, bs2):
    f32 = jnp.float32
    return pl.pallas_call(
        _tc_pre2_body,
        grid=(NB,),
        in_specs=[_row_spec(HID), _const_spec((8, HID)), _const_spec((HID,)),
                  _const_spec((HID,)), _row_spec(16), _const_spec((HID, 264)),
                  _const_spec((HID,))],
        out_specs=(_row_spec(WT), _row_spec(16), _row_spec(16),
                   _row_spec(HID)),
        out_shape=(jax.ShapeDtypeStruct((N, WT), f32),
                   jax.ShapeDtypeStruct((N, 16), f32),
                   jax.ShapeDtypeStruct((N, 16), f32),
                   jax.ShapeDtypeStruct((N, HID), f32)),
    )(t, stats, g1, be1, a2, wcat2, bs2)


def _tc_post2_body(p0_ref, p1_ref, htab_ref, ps_ref, skip_ref, b2_ref,
                   wf_ref, bf_ref, out_ref):
    y = _combine(p0_ref[...], p1_ref[...], htab_ref[...], ps_ref[...],
                 skip_ref[...], b2_ref[...])
    y = jnp.where(y > 0, y, jnp.exp(y) - 1.0)
    out_ref[...] = jnp.dot(y, wf_ref[...], preferred_element_type=jnp.float32) \
        + bf_ref[...][None, :]


@jax.jit
def _tc_post2(parts, htab, pself, skip, b2, wf, bf):
    return pl.pallas_call(
        _tc_post2_body,
        grid=(NB,),
        in_specs=[_row_spec(WT), _row_spec(WT), _row_spec(WT), _row_spec(16),
                  _row_spec(HID), _const_spec((HID,)), _const_spec((HID, 1)),
                  _const_spec((1,))],
        out_specs=pl.BlockSpec((BN_, 1), lambda i: (i, 0)),
        out_shape=jax.ShapeDtypeStruct((N, 1), jnp.float32),
    )(parts[0], parts[1], htab, pself, skip, b2, wf, bf)


def kernel(x, edge_attr, edge_index, W1, att_src1, att_dst1, We1, att_e1, b1,
           Ws1, bs1, g1, be1, W2, att_src2, att_dst2, We2, att_e2, b2, Ws2,
           bs2, Wf, bf):
    f32 = jnp.float32
    src = edge_index[0].astype(jnp.int32)
    dst = edge_index[1].astype(jnp.int32)

    def proj(W, a_s, a_d, We, a_e, din):
        wsrc = (W.reshape(din, H, O) * a_s[None]).sum(-1)
        wdst = (W.reshape(din, H, O) * a_d[None]).sum(-1)
        weproj = (We.reshape(DE, H, O) * a_e[None]).sum(-1)
        return wsrc, wdst, weproj

    wsrc1, wdst1, weproj1 = proj(W1, att_src1, att_dst1, We1, att_e1, D)
    wsrc2, wdst2, weproj2 = proj(W2, att_src2, att_dst2, We2, att_e2, HID)
    wcat1 = jnp.concatenate([W1, Ws1, wsrc1, wdst1], axis=1).astype(f32)
    wcat2 = jnp.concatenate([W2, Ws2, wsrc2, wdst2], axis=1).astype(f32)

    ea32 = edge_attr.astype(f32)
    htab1, dtab1, skip1 = _tc_pre1(x.astype(f32), wcat1, bs1)
    zc = jnp.zeros((DE, 1), f32)
    wp16 = jnp.concatenate([weproj1, weproj1, zc, weproj2, zc, zc, zc], axis=1)
    wp2p = jnp.concatenate([weproj2] + [zc] * 12, axis=1)
    ey8 = jnp.eye(8, dtype=f32)
    ae12, ae2 = _tc_ae(ea32.reshape(E // 8, 128),
                       jnp.kron(ey8, wp16), jnp.kron(ey8, wp2p))
    parts1 = _sc_messages(htab1, dtab1, ae12, src, dst)
    t, stats, a2 = _tc_combine1(parts1, htab1, dtab1, skip1, b1)
    htab2, dtab2, ps2, skip2 = _tc_pre2(t, stats, g1, be1, a2, wcat2, bs2)
    parts2 = _sc_messages(htab2, dtab2, ae2, src, dst)
    return _tc_post2(parts2, htab2, ps2, skip2, b2, Wf, bf)


# speedup vs baseline: 84.7325x; 1.0225x over previous
import jax
import jax.numpy as jnp
from jax import lax
from jax.experimental import pallas as pl
from jax.experimental.pallas import tpu as pltpu
from jax.experimental.pallas import tpu_sc as plsc

N = 10000
E = 320000
D = 128
DE = 16
H = 4
O = 32
HID = H * O

NC = 2
NS = 16
NW = NC * NS
EW = E // NW
CH = 80
NCHUNK = EW // CH
NP = 10240
RPT = NP // NS
WT = 144

_mesh = plsc.VectorSubcoreMesh(core_axis_name="c", subcore_axis_name="s",
                               num_cores=NC, num_subcores=NS)
_sc_params = pltpu.CompilerParams(use_tc_tiling_on_sc=False)
_tc_params = pltpu.CompilerParams(vmem_limit_bytes=100 * 1024 * 1024)


def _zero_rows(zbuf, width):
    zv = jnp.zeros((16,), jnp.float32)

    @pl.loop(0, zbuf.shape[0])
    def _(i):
        for j in range(width // 16):
            zbuf[i, pl.ds(16 * j, 16)] = zv


SB = 25
NSB = NCHUNK // SB


def _sc_messages_body(htab_hbm, dtab_hbm, ae_hbm, src_hbm, dst_hbm,
                      outm_hbm, outa_hbm,
                      acc, gsrc0, gsrc1, gdst0, gdst1, aeb0, aeb1,
                      idxs_sb, idxd_sb, sem0, sem1):
    c = lax.axis_index("c")
    s = lax.axis_index("s")
    wid = c * NS + s

    _zero_rows(gsrc0, WT)
    for k in range(RPT // CH):
        r0 = s * RPT + k * CH
        pltpu.sync_copy(gsrc0, acc.at[pl.ds(r0, CH)])
    plsc.subcore_barrier()

    hidx = [jnp.full((16, 1), h, jnp.int32) for h in range(H)]
    gdn = lax.GatherDimensionNumbers(offset_dims=(), collapsed_slice_dims=(0,),
                                     start_index_map=(0,))

    def _splat(vec, idx):
        return lax.gather(vec, idx, gdn, (1,),
                          mode=lax.GatherScatterMode.PROMISE_IN_BOUNDS)

    lane = lax.iota(jnp.int32, 16)
    lt4 = jnp.where(lane < 4, 1.0, 0.0).astype(jnp.float32)
    lt8 = jnp.where(lane < 8, 1.0, 0.0).astype(jnp.float32)
    lt9 = jnp.where(lane < 9, 1.0, 0.0).astype(jnp.float32)
    lt13 = jnp.where(lane < 13, 1.0, 0.0).astype(jnp.float32)
    m1 = lt4 + (lt9 - lt8)
    m2 = (lt8 - lt4) + (lt13 - lt9)

    bufs = ((gsrc0, gdst0, aeb0, sem0), (gsrc1, gdst1, aeb1, sem1))
    CR = CH // 8

    def issue(sstep, cl, b):
        gsrc, gdst, aeb, sem = bufs[b]
        base = wid * EW + sstep * (SB * CH) + cl * CH
        pltpu.async_copy(ae_hbm.at[pl.ds(base // 8, CR)], aeb, sem)
        pltpu.async_copy(htab_hbm.at[idxs_sb.at[cl]], gsrc, sem)
        pltpu.async_copy(dtab_hbm.at[idxd_sb.at[cl]], gdst, sem)

    def consume(cl, b):
        gsrc, gdst, aeb, sem = bufs[b]
        pltpu.make_async_copy(ae_hbm.at[pl.ds(0, CR)], aeb, sem).wait()
        pltpu.make_async_copy(htab_hbm.at[idxs_sb.at[cl]], gsrc, sem).wait()
        pltpu.make_async_copy(dtab_hbm.at[idxd_sb.at[cl]], gdst, sem).wait()

        @pl.loop(0, CR, unroll=2)
        def _(r):
            for j in range(8):
                e = r * 8 + j
                av = aeb[r, pl.ds(16 * j, 16)]
                t = gsrc[e, pl.ds(128, 16)] + gdst[e, pl.ds(0, 16)] + av
                t = jnp.maximum(t, 0.0) + 0.2 * jnp.minimum(t, 0.0)
                p = jnp.exp(t)
                gsrc[e, pl.ds(128, 16)] = p * m1 + av * m2
                for h in range(H):
                    pb = _splat(p, hidx[h])
                    gsrc[e, pl.ds(32 * h, 16)] = gsrc[e, pl.ds(32 * h, 16)] * pb
                    gsrc[e, pl.ds(32 * h + 16, 16)] = \
                        gsrc[e, pl.ds(32 * h + 16, 16)] * pb

        pltpu.sync_copy(gsrc, acc.at[idxd_sb.at[cl]], add=True)

    @pl.loop(0, NSB)
    def _(sstep):
        row0 = wid * NCHUNK + sstep * SB
        pltpu.sync_copy(src_hbm.at[pl.ds(row0, SB)], idxs_sb)
        pltpu.sync_copy(dst_hbm.at[pl.ds(row0, SB)], idxd_sb)
        issue(sstep, 0, 0)

        @pl.loop(0, (SB - 1) // 2)
        def _(j):
            issue(sstep, 2 * j + 1, 1)
            consume(2 * j, 0)
            issue(sstep, 2 * j + 2, 0)
            consume(2 * j + 1, 1)

        consume(SB - 1, 0)

    plsc.subcore_barrier()
    r0 = s * RPT
    pltpu.sync_copy(acc.at[pl.ds(r0, RPT), pl.ds(0, 128)],
                    outm_hbm.at[c].at[pl.ds(r0, RPT)])
    pltpu.sync_copy(acc.at[pl.ds(r0, RPT), pl.ds(128, 16)],
                    outa_hbm.at[c].at[pl.ds(r0, RPT)])


@jax.jit
def _sc_messages(htab, dtab, ae_pad, src, dst):
    f32 = jnp.float32
    kern = pl.kernel(
        _sc_messages_body,
        out_type=(jax.ShapeDtypeStruct((NC, NP, 128), f32),
                  jax.ShapeDtypeStruct((NC, NP, 16), f32)),
        mesh=_mesh,
        scratch_types=[
            pltpu.VMEM_SHARED((NP, WT), f32),
            pltpu.VMEM((CH, WT), f32),
            pltpu.VMEM((CH, WT), f32),
            pltpu.VMEM((CH, 16), f32),
            pltpu.VMEM((CH, 16), f32),
            pltpu.VMEM((CH // 8, 128), f32),
            pltpu.VMEM((CH // 8, 128), f32),
            pltpu.VMEM((SB, CH), jnp.int32),
            pltpu.VMEM((SB, CH), jnp.int32),
            pltpu.SemaphoreType.DMA,
            pltpu.SemaphoreType.DMA,
        ],
        compiler_params=_sc_params,
    )
    return kern(htab, dtab, ae_pad, src.reshape(NW * NCHUNK, CH),
                dst.reshape(NW * NCHUNK, CH))


BN_ = 2000
NB = N // BN_


def _lrelu(x):
    return jnp.maximum(x, 0.0) + 0.2 * jnp.minimum(x, 0.0)


def _rep32(a):
    n = a.shape[0]
    return jnp.broadcast_to(a[:, :, None], (n, H, O)).reshape(n, HID)


def _node_tables(big, bs, ael):
    n = big.shape[0]
    h = big[:, :HID]
    skip = big[:, HID:2 * HID] + bs[None, :]
    a_s = big[:, 2 * HID:2 * HID + H]
    a_d = big[:, 2 * HID + H:2 * HID + 2 * H]
    p_self = jnp.exp(_lrelu(a_s + a_d + ael))
    z12 = jnp.zeros((n, 12), jnp.float32)
    htab = jnp.concatenate([h, a_s, z12], axis=1)
    dtab = jnp.concatenate([a_d, z12], axis=1)
    pself = jnp.concatenate([p_self, z12], axis=1)
    return htab, dtab, pself, skip


def _row_spec(w):
    return pl.BlockSpec((BN_, w), lambda i: (i, 0))


def _const_spec(shape):
    nd = len(shape)
    return pl.BlockSpec(shape, lambda i: (0,) * nd)


def _tc_pre1_body(x_ref, wcat_ref, bs_ref, htab_ref, dtab_ref, skip_ref):
    big = jnp.dot(x_ref[...], wcat_ref[...], preferred_element_type=jnp.float32)
    n = big.shape[0]
    z12 = jnp.zeros((n, 12), jnp.float32)
    htab_ref[...] = jnp.concatenate(
        [big[:, :HID], big[:, 2 * HID:2 * HID + H], z12], axis=1)
    dtab_ref[...] = jnp.concatenate(
        [big[:, 2 * HID + H:2 * HID + 2 * H], z12], axis=1)
    skip_ref[...] = big[:, HID:2 * HID] + bs_ref[...][None, :]


@jax.jit
def _tc_pre1(x, wcat, bs):
    f32 = jnp.float32
    return pl.pallas_call(
        _tc_pre1_body,
        grid=(NB,),
        in_specs=[_row_spec(D), _const_spec((D, 264)), _const_spec((HID,))],
        out_specs=(_row_spec(WT), _row_spec(16), _row_spec(HID)),
        out_shape=(jax.ShapeDtypeStruct((N, WT), f32),
                   jax.ShapeDtypeStruct((N, 16), f32),
                   jax.ShapeDtypeStruct((N, HID), f32)),
    )(x, wcat, bs)


def _tc_ae_body(ea_ref, wpa_ref, wpb_ref, outa_ref, outb_ref):
    ea = ea_ref[...]
    outa_ref[...] = jnp.dot(ea, wpa_ref[...],
                            preferred_element_type=jnp.float32)
    outb_ref[...] = jnp.dot(ea, wpb_ref[...],
                            preferred_element_type=jnp.float32)


@jax.jit
def _tc_ae(ea128, wpa, wpb):
    br = 4000
    spec = pl.BlockSpec((br, 128), lambda i: (i, 0))
    shp = jax.ShapeDtypeStruct((E // 8, 128), jnp.float32)
    return pl.pallas_call(
        _tc_ae_body,
        grid=(E // 8 // br,),
        in_specs=[spec, _const_spec((128, 128)), _const_spec((128, 128))],
        out_specs=(spec, spec),
        out_shape=(shp, shp),
    )(ea128, wpa, wpb)


def _combine(msg, den, htab, pself, skip, b):
    p_self = pself[:, :H]
    hmat = htab[:, :HID]
    dent = den + p_self
    x1 = (msg + hmat * _rep32(p_self)) / _rep32(dent + 1e-16) + b[None, :]
    return x1 + skip


def _tc_combine1_body(pm0_ref, pm1_ref, pa0_ref, pa1_ref, htab_ref, dtab_ref,
                      skip_ref, b1_ref, t_ref, stats_ref, a2_ref):
    P = pm0_ref[...] + pm1_ref[...]
    A = pa0_ref[...] + pa1_ref[...]
    htab = htab_ref[...]
    n = P.shape[0]
    deg = jnp.maximum(A[:, 8:9], 1.0)
    ael1 = A[:, 4:8] / deg
    a2_ref[...] = jnp.concatenate(
        [A[:, 9:13] / deg, jnp.zeros((n, 12), jnp.float32)], axis=1)
    a_s = htab[:, HID:HID + H]
    a_d = dtab_ref[...][:, :H]
    p_self = jnp.exp(_lrelu(a_s + a_d + ael1))
    den = A[:, :H] + p_self
    t = (P + htab[:, :HID] * _rep32(p_self)) \
        / _rep32(den + 1e-16) + b1_ref[...][None, :] + skip_ref[...]
    t_ref[...] = t
    s1 = jnp.sum(t, axis=0, keepdims=True)
    s2 = jnp.sum(t * t, axis=0, keepdims=True)
    part = jnp.concatenate([s1, s2, jnp.zeros((6, HID), jnp.float32)], axis=0)

    @pl.when(pl.program_id(0) == 0)
    def _():
        stats_ref[...] = jnp.zeros((8, HID), jnp.float32)

    stats_ref[...] += part


@jax.jit
def _tc_combine1(pm, pa, htab, dtab, skip, b1):
    f32 = jnp.float32
    return pl.pallas_call(
        _tc_combine1_body,
        grid=(NB,),
        in_specs=[_row_spec(128), _row_spec(128), _row_spec(16), _row_spec(16),
                  _row_spec(WT), _row_spec(16), _row_spec(HID),
                  _const_spec((HID,))],
        out_specs=(_row_spec(HID), _const_spec((8, HID)), _row_spec(16)),
        out_shape=(jax.ShapeDtypeStruct((N, HID), f32),
                   jax.ShapeDtypeStruct((8, HID), f32),
                   jax.ShapeDtypeStruct((N, 16), f32)),
    )(pm[0], pm[1], pa[0], pa[1], htab, dtab, skip, b1)


def _tc_pre2_body(t_ref, stats_ref, g_ref, be_ref, a2_ref, wcat_ref,
                  bs_ref, htab_ref, dtab_ref, ps_ref, skip_ref):
    t = t_ref[...]
    mu = stats_ref[0:1, :] * (1.0 / N)
    var = stats_ref[1:2, :] * (1.0 / N) - mu * mu
    hn = (t - mu) / jnp.sqrt(var + 1e-5) * g_ref[...][None, :] \
        + be_ref[...][None, :]
    h = jnp.where(hn > 0, hn, jnp.exp(hn) - 1.0)
    big = jnp.dot(h, wcat_ref[...], preferred_element_type=jnp.float32)
    ael = a2_ref[...][:, :H]
    htab, dtab, pself, skip = _node_tables(big, bs_ref[...], ael)
    htab_ref[...] = htab
    dtab_ref[...] = dtab
    ps_ref[...] = pself
    skip_ref[...] = skip


@jax.jit
def _tc_pre2(t, stats, g1, be1, a2, wcat2, bs2):
    f32 = jnp.float32
    return pl.pallas_call(
        _tc_pre2_body,
        grid=(NB,),
        in_specs=[_row_spec(HID), _const_spec((8, HID)), _const_spec((HID,)),
                  _const_spec((HID,)), _row_spec(16), _const_spec((HID, 264)),
                  _const_spec((HID,))],
        out_specs=(_row_spec(WT), _row_spec(16), _row_spec(16),
                   _row_spec(HID)),
        out_shape=(jax.ShapeDtypeStruct((N, WT), f32),
                   jax.ShapeDtypeStruct((N, 16), f32),
                   jax.ShapeDtypeStruct((N, 16), f32),
                   jax.ShapeDtypeStruct((N, HID), f32)),
    )(t, stats, g1, be1, a2, wcat2, bs2)


def _tc_post2_body(pm0_ref, pm1_ref, pa0_ref, pa1_ref, htab_ref, ps_ref,
                   skip_ref, b2_ref, wf_ref, bf_ref, out_ref):
    msg = pm0_ref[...] + pm1_ref[...]
    den = pa0_ref[...][:, :H] + pa1_ref[...][:, :H]
    y = _combine(msg, den, htab_ref[...], ps_ref[...], skip_ref[...],
                 b2_ref[...])
    y = jnp.where(y > 0, y, jnp.exp(y) - 1.0)
    out_ref[...] = jnp.dot(y, wf_ref[...], preferred_element_type=jnp.float32) \
        + bf_ref[...][None, :]


@jax.jit
def _tc_post2(pm, pa, htab, pself, skip, b2, wf, bf):
    return pl.pallas_call(
        _tc_post2_body,
        grid=(NB,),
        in_specs=[_row_spec(128), _row_spec(128), _row_spec(16), _row_spec(16),
                  _row_spec(WT), _row_spec(16), _row_spec(HID),
                  _const_spec((HID,)), _const_spec((HID, 1)),
                  _const_spec((1,))],
        out_specs=pl.BlockSpec((BN_, 1), lambda i: (i, 0)),
        out_shape=jax.ShapeDtypeStruct((N, 1), jnp.float32),
    )(pm[0], pm[1], pa[0], pa[1], htab, pself, skip, b2, wf, bf)


def kernel(x, edge_attr, edge_index, W1, att_src1, att_dst1, We1, att_e1, b1,
           Ws1, bs1, g1, be1, W2, att_src2, att_dst2, We2, att_e2, b2, Ws2,
           bs2, Wf, bf):
    f32 = jnp.float32
    src = edge_index[0].astype(jnp.int32)
    dst = edge_index[1].astype(jnp.int32)

    def proj(W, a_s, a_d, We, a_e, din):
        wsrc = (W.reshape(din, H, O) * a_s[None]).sum(-1)
        wdst = (W.reshape(din, H, O) * a_d[None]).sum(-1)
        weproj = (We.reshape(DE, H, O) * a_e[None]).sum(-1)
        return wsrc, wdst, weproj

    wsrc1, wdst1, weproj1 = proj(W1, att_src1, att_dst1, We1, att_e1, D)
    wsrc2, wdst2, weproj2 = proj(W2, att_src2, att_dst2, We2, att_e2, HID)
    wcat1 = jnp.concatenate([W1, Ws1, wsrc1, wdst1], axis=1).astype(f32)
    wcat2 = jnp.concatenate([W2, Ws2, wsrc2, wdst2], axis=1).astype(f32)

    ea32 = edge_attr.astype(f32)
    htab1, dtab1, skip1 = _tc_pre1(x.astype(f32), wcat1, bs1)
    zc = jnp.zeros((DE, 1), f32)
    wp16 = jnp.concatenate([weproj1, weproj1, zc, weproj2, zc, zc, zc], axis=1)
    wp2p = jnp.concatenate([weproj2] + [zc] * 12, axis=1)
    ey8 = jnp.eye(8, dtype=f32)
    ae12, ae2 = _tc_ae(ea32.reshape(E // 8, 128),
                       jnp.kron(ey8, wp16), jnp.kron(ey8, wp2p))
    pm1, pa1 = _sc_messages(htab1, dtab1, ae12, src, dst)
    t, stats, a2 = _tc_combine1(pm1, pa1, htab1, dtab1, skip1, b1)
    htab2, dtab2, ps2, skip2 = _tc_pre2(t, stats, g1, be1, a2, wcat2, bs2)
    pm2, pa2 = _sc_messages(htab2, dtab2, ae2, src, dst)
    return _tc_post2(pm2, pa2, htab2, ps2, skip2, b2, Wf, bf)
